# baseline (speedup 1.0000x reference)
import functools

import jax
import jax.numpy as jnp
from jax import lax
from jax.experimental import pallas as pl
from jax.experimental.pallas import tpu as pltpu
from jax.experimental.pallas import tpu_sc as plsc

N = 10000
E = 320000
H = 128
ED = 16
CUTOFF = 5.0

_NC = 2
_NS = 16
_NW = _NC * _NS
_CH = 128
_ECH = E // _CH
_RPE = _ECH // _NW
_XW = _ECH - _RPE * _NW
_K = 6
_NG = _RPE // _K
_RW = 624
_RREM = N - _NS * _RW
_ROFF = _NS * _RW

_mesh = plsc.VectorSubcoreMesh(core_axis_name="c", subcore_axis_name="s")
_f32 = jnp.float32


def _zero_vmem(ref, nrows, ncols):
    nv = ncols // 16

    def body(i, _):
        for j in range(nv):
            ref[i, pl.ds(j * 16, 16)] = jnp.zeros((16,), _f32)
        return 0

    lax.fori_loop(0, nrows, body, 0)


def _zero_shared_slice(zbuf, acc_sh, sid):
    r0 = pl.multiple_of(sid * _RW, 8)
    nfull = _RW // 128
    rem = _RW - nfull * 128
    for k in range(nfull):
        pltpu.sync_copy(zbuf.at[pl.ds(0, 128)],
                        acc_sh.at[pl.ds(r0 + k * 128, 128)])
    if rem:
        pltpu.sync_copy(zbuf.at[pl.ds(0, rem)],
                        acc_sh.at[pl.ds(r0 + nfull * 128, rem)])

    @pl.when(sid == _NS - 1)
    def _():
        pltpu.sync_copy(zbuf.at[pl.ds(0, _RREM)], acc_sh.at[pl.ds(_ROFF, _RREM)])


def _writeback_slice(acc_sh, out_hbm, cid, sid):
    r0 = pl.multiple_of(sid * _RW, 8)
    pltpu.sync_copy(acc_sh.at[pl.ds(r0, _RW)], out_hbm.at[cid, pl.ds(r0, _RW)])

    @pl.when(sid == _NS - 1)
    def _():
        pltpu.sync_copy(acc_sh.at[pl.ds(_ROFF, _RREM)],
                        out_hbm.at[cid, pl.ds(_ROFF, _RREM)])


_KA = 2
_SB = 26
_NSB = _RPE // _SB
_NGA = _SB // _KA


@functools.partial(
    pl.kernel,
    out_type=jax.ShapeDtypeStruct((_NC, N, H), _f32),
    mesh=_mesh,
    scratch_types=[
        pltpu.VMEM((_SB, _CH), jnp.int32),
        pltpu.VMEM((_SB, _CH), jnp.int32),
        pltpu.VMEM((_KA * _CH, H), _f32),
        pltpu.VMEM_SHARED((N, H), _f32),
        pltpu.SemaphoreType.DMA,
        pltpu.SemaphoreType.DMA,
    ],
    compiler_params=pltpu.CompilerParams(use_tc_tiling_on_sc=False),
)
def _agg_sc(h_hbm, row2_hbm, col2_hbm, out_hbm,
            ridxb, cidxb, rows, agg_sh, gsem, ssem):
    cid = lax.axis_index("c")
    sid = lax.axis_index("s")
    w = cid * _NS + sid
    rb = w * _RPE

    _zero_vmem(rows, 128, H)
    _zero_shared_slice(rows, agg_sh, sid)
    plsc.subcore_barrier()

    def superblock(sb, _):
        pltpu.sync_copy(row2_hbm.at[pl.ds(rb + sb * _SB, _SB)], ridxb)
        pltpu.sync_copy(col2_hbm.at[pl.ds(rb + sb * _SB, _SB)], cidxb)

        def group(g, _):
            gd = [pltpu.async_copy(h_hbm.at[ridxb.at[g * _KA + k]],
                                   rows.at[pl.ds(k * _CH, _CH)], gsem)
                  for k in range(_KA)]
            for d in gd:
                d.wait()
            sd = [pltpu.async_copy(rows.at[pl.ds(k * _CH, _CH)],
                                   agg_sh.at[cidxb.at[g * _KA + k]], ssem,
                                   add=True)
                  for k in range(_KA)]
            for d in sd:
                d.wait()
            return 0

        lax.fori_loop(0, _NGA, group, 0)
        return 0

    lax.fori_loop(0, _NSB, superblock, 0)

    @pl.when(w < _XW)
    def _():
        pltpu.sync_copy(row2_hbm.at[pl.ds(_NW * _RPE + w, 1)],
                        ridxb.at[pl.ds(0, 1)])
        pltpu.sync_copy(col2_hbm.at[pl.ds(_NW * _RPE + w, 1)],
                        cidxb.at[pl.ds(0, 1)])
        pltpu.async_copy(h_hbm.at[ridxb.at[0]],
                         rows.at[pl.ds(0, _CH)], gsem).wait()
        pltpu.async_copy(rows.at[pl.ds(0, _CH)],
                         agg_sh.at[cidxb.at[0]], ssem, add=True).wait()

    plsc.subcore_barrier()
    _writeback_slice(agg_sh, out_hbm, cid, sid)


@functools.partial(
    pl.kernel,
    out_type=[jax.ShapeDtypeStruct((_NC, N, H), _f32),
              jax.ShapeDtypeStruct((E, ED), _f32),
              jax.ShapeDtypeStruct((E, ED), _f32)],
    mesh=_mesh,
    scratch_types=[
        pltpu.VMEM((_SB, _CH), jnp.int32),
        pltpu.VMEM((_SB, _CH), jnp.int32),
        pltpu.VMEM((_KA * _CH, H), _f32),
        pltpu.VMEM((_KA * _CH, ED), _f32),
        pltpu.VMEM((_KA * _CH, ED), _f32),
        pltpu.VMEM_SHARED((N, H), _f32),
        pltpu.SemaphoreType.DMA,
        pltpu.SemaphoreType.DMA,
        pltpu.SemaphoreType.DMA,
    ],
    compiler_params=pltpu.CompilerParams(use_tc_tiling_on_sc=False),
)
def _aggpair_sc(h_hbm, ta_hbm, tb_hbm, row2_hbm, col2_hbm,
                out_hbm, oa_hbm, ob_hbm,
                ridxb, cidxb, rows, pbufa, pbufb, agg_sh, gsem, ssem, wsem):
    cid = lax.axis_index("c")
    sid = lax.axis_index("s")
    w = cid * _NS + sid
    rb = w * _RPE

    _zero_vmem(rows, 128, H)
    _zero_shared_slice(rows, agg_sh, sid)
    plsc.subcore_barrier()

    def superblock(sb, _):
        pltpu.sync_copy(row2_hbm.at[pl.ds(rb + sb * _SB, _SB)], ridxb)
        pltpu.sync_copy(col2_hbm.at[pl.ds(rb + sb * _SB, _SB)], cidxb)

        def group(g, _):
            gd = [pltpu.async_copy(h_hbm.at[ridxb.at[g * _KA + k]],
                                   rows.at[pl.ds(k * _CH, _CH)], gsem)
                  for k in range(_KA)]
            gd += [pltpu.async_copy(ta_hbm.at[ridxb.at[g * _KA + k]],
                                    pbufa.at[pl.ds(k * _CH, _CH)], gsem)
                   for k in range(_KA)]
            gd += [pltpu.async_copy(tb_hbm.at[cidxb.at[g * _KA + k]],
                                    pbufb.at[pl.ds(k * _CH, _CH)], gsem)
                   for k in range(_KA)]
            for d in gd:
                d.wait()
            sd = [pltpu.async_copy(rows.at[pl.ds(k * _CH, _CH)],
                                   agg_sh.at[cidxb.at[g * _KA + k]], ssem,
                                   add=True)
                  for k in range(_KA)]
            e0 = (rb + sb * _SB + g * _KA) * _CH
            wa = pltpu.async_copy(pbufa, oa_hbm.at[pl.ds(e0, _KA * _CH)], wsem)
            wb = pltpu.async_copy(pbufb, ob_hbm.at[pl.ds(e0, _KA * _CH)], wsem)
            for d in sd:
                d.wait()
            wa.wait()
            wb.wait()
            return 0

        lax.fori_loop(0, _NGA, group, 0)
        return 0

    lax.fori_loop(0, _NSB, superblock, 0)

    @pl.when(w < _XW)
    def _():
        pltpu.sync_copy(row2_hbm.at[pl.ds(_NW * _RPE + w, 1)],
                        ridxb.at[pl.ds(0, 1)])
        pltpu.sync_copy(col2_hbm.at[pl.ds(_NW * _RPE + w, 1)],
                        cidxb.at[pl.ds(0, 1)])
        gd = [pltpu.async_copy(h_hbm.at[ridxb.at[0]],
                               rows.at[pl.ds(0, _CH)], gsem),
              pltpu.async_copy(ta_hbm.at[ridxb.at[0]],
                               pbufa.at[pl.ds(0, _CH)], gsem),
              pltpu.async_copy(tb_hbm.at[cidxb.at[0]],
                               pbufb.at[pl.ds(0, _CH)], gsem)]
        for d in gd:
            d.wait()
        e0 = (_NW * _RPE + w) * _CH
        sd = [pltpu.async_copy(rows.at[pl.ds(0, _CH)],
                               agg_sh.at[cidxb.at[0]], ssem, add=True),
              pltpu.async_copy(pbufa.at[pl.ds(0, _CH)],
                               oa_hbm.at[pl.ds(e0, _CH)], wsem),
              pltpu.async_copy(pbufb.at[pl.ds(0, _CH)],
                               ob_hbm.at[pl.ds(e0, _CH)], wsem)]
        for d in sd:
            d.wait()

    plsc.subcore_barrier()
    _writeback_slice(agg_sh, out_hbm, cid, sid)


@functools.partial(
    pl.kernel,
    out_type=[jax.ShapeDtypeStruct((E, ED), _f32),
              jax.ShapeDtypeStruct((E, ED), _f32)],
    mesh=_mesh,
    scratch_types=[
        pltpu.VMEM((_RPE + 1, _CH), jnp.int32),
        pltpu.VMEM((_RPE + 1, _CH), jnp.int32),
        pltpu.VMEM((_K * _CH, ED), _f32),
        pltpu.VMEM((_K * _CH, ED), _f32),
        pltpu.SemaphoreType.DMA,
        pltpu.SemaphoreType.DMA,
    ],
    compiler_params=pltpu.CompilerParams(use_tc_tiling_on_sc=False),
)
def _pair_sc(ta_hbm, tb_hbm, ia2_hbm, ib2_hbm, oa_hbm, ob_hbm,
             idxa2, idxb2, bufa, bufb, gsem, wsem):
    cid = lax.axis_index("c")
    sid = lax.axis_index("s")
    w = cid * _NS + sid
    rb = w * _RPE
    base = rb * _CH

    pltpu.sync_copy(ia2_hbm.at[pl.ds(rb, _RPE)], idxa2.at[pl.ds(0, _RPE)])
    pltpu.sync_copy(ib2_hbm.at[pl.ds(rb, _RPE)], idxb2.at[pl.ds(0, _RPE)])

    @pl.when(w < _XW)
    def _():
        pltpu.sync_copy(ia2_hbm.at[pl.ds(_NW * _RPE + w, 1)],
                        idxa2.at[pl.ds(_RPE, 1)])
        pltpu.sync_copy(ib2_hbm.at[pl.ds(_NW * _RPE + w, 1)],
                        idxb2.at[pl.ds(_RPE, 1)])

    def group(g, _):
        gd = [pltpu.async_copy(ta_hbm.at[idxa2.at[g * _K + k]],
                               bufa.at[pl.ds(k * _CH, _CH)], gsem)
              for k in range(_K)]
        gd += [pltpu.async_copy(tb_hbm.at[idxb2.at[g * _K + k]],
                                bufb.at[pl.ds(k * _CH, _CH)], gsem)
               for k in range(_K)]
        for d in gd:
            d.wait()
        e0 = base + g * _K * _CH
        wa = pltpu.async_copy(bufa, oa_hbm.at[pl.ds(e0, _K * _CH)], wsem)
        wb = pltpu.async_copy(bufb, ob_hbm.at[pl.ds(e0, _K * _CH)], wsem)
        wa.wait()
        wb.wait()
        return 0

    lax.fori_loop(0, _NG, group, 0)

    @pl.when(w < _XW)
    def _():
        ga = pltpu.async_copy(ta_hbm.at[idxa2.at[_RPE]],
                              bufa.at[pl.ds(0, _CH)], gsem)
        gb = pltpu.async_copy(tb_hbm.at[idxb2.at[_RPE]],
                              bufb.at[pl.ds(0, _CH)], gsem)
        ga.wait()
        gb.wait()
        e0 = (_NW * _RPE + w) * _CH
        wa = pltpu.async_copy(bufa.at[pl.ds(0, _CH)],
                              oa_hbm.at[pl.ds(e0, _CH)], wsem)
        wb = pltpu.async_copy(bufb.at[pl.ds(0, _CH)],
                              ob_hbm.at[pl.ds(e0, _CH)], wsem)
        wa.wait()
        wb.wait()


@functools.partial(
    pl.kernel,
    out_type=jax.ShapeDtypeStruct((_NC, N, ED), _f32),
    mesh=_mesh,
    scratch_types=[
        pltpu.VMEM((_RPE + 1, _CH), jnp.int32),
        pltpu.VMEM((_K * _CH, ED), _f32),
        pltpu.VMEM_SHARED((N, ED), _f32),
        pltpu.SemaphoreType.DMA,
        pltpu.SemaphoreType.DMA,
    ],
    compiler_params=pltpu.CompilerParams(use_tc_tiling_on_sc=False),
)
def _scatter16_sc(vals_hbm, col2_hbm, out_hbm, cidx2, vbuf, acc_sh, lsem, ssem):
    cid = lax.axis_index("c")
    sid = lax.axis_index("s")
    w = cid * _NS + sid
    rb = w * _RPE
    base = rb * _CH

    pltpu.sync_copy(col2_hbm.at[pl.ds(rb, _RPE)], cidx2.at[pl.ds(0, _RPE)])

    @pl.when(w < _XW)
    def _():
        pltpu.sync_copy(col2_hbm.at[pl.ds(_NW * _RPE + w, 1)],
                        cidx2.at[pl.ds(_RPE, 1)])

    _zero_vmem(vbuf, 128, ED)
    _zero_shared_slice(vbuf, acc_sh, sid)
    plsc.subcore_barrier()

    def group(g, _):
        e0 = base + g * _K * _CH
        pltpu.async_copy(vals_hbm.at[pl.ds(e0, _K * _CH)], vbuf, lsem).wait()
        sd = [pltpu.async_copy(vbuf.at[pl.ds(k * _CH, _CH)],
                               acc_sh.at[cidx2.at[g * _K + k]], ssem, add=True)
              for k in range(_K)]
        for d in sd:
            d.wait()
        return 0

    lax.fori_loop(0, _NG, group, 0)

    @pl.when(w < _XW)
    def _():
        e0 = (_NW * _RPE + w) * _CH
        pltpu.async_copy(vals_hbm.at[pl.ds(e0, _CH)],
                         vbuf.at[pl.ds(0, _CH)], lsem).wait()
        pltpu.async_copy(vbuf.at[pl.ds(0, _CH)],
                         acc_sh.at[cidx2.at[_RPE]], ssem, add=True).wait()

    plsc.subcore_barrier()
    _writeback_slice(acc_sh, out_hbm, cid, sid)


_NB = 1000
_GN = N // _NB
_BE = 8000
_GE = E // _BE


def _full(shape):
    return pl.BlockSpec(shape, lambda i: tuple(0 for _ in shape))


def _rows(shape):
    return pl.BlockSpec(shape, lambda i: (i,) + tuple(0 for _ in shape[1:]))


def _embed_body(x_ref, w_ref, b_ref, o_ref):
    o_ref[...] = jnp.dot(x_ref[...], w_ref[...],
                         preferred_element_type=_f32) + b_ref[...]


def _embed(x, w, b):
    return pl.pallas_call(
        _embed_body,
        grid=(_GN,),
        in_specs=[_rows((_NB, H)), _full((H, H)), _full((1, H))],
        out_specs=_rows((_NB, H)),
        out_shape=jax.ShapeDtypeStruct((N, H), _f32),
    )(x, w, b.reshape(1, H))


def _gauss_body(d_ref, off_ref, g_ref, o_ref):
    d = d_ref[...]
    o_ref[...] = jnp.exp(g_ref[0, 0] * (d - off_ref[...]) ** 2)


def _gauss(edge_attr, offs, gamma):
    return pl.pallas_call(
        _gauss_body,
        grid=(_GE,),
        in_specs=[_rows((_BE, 1)), _full((1, ED)), _full((1, 1))],
        out_specs=_rows((_BE, ED)),
        out_shape=jax.ShapeDtypeStruct((E, ED), _f32),
    )(edge_attr.reshape(E, 1), offs.reshape(1, ED), gamma.reshape(1, 1))


def _silu(v):
    return v * jax.nn.sigmoid(v)


def _node_body(h_ref, a0_ref, a1_ref, w1h_ref, w1a_ref, b1_ref, w2_ref,
               b2_ref, g_ref, bb_ref, wr_ref, wc_ref,
               hn_ref, pr_ref, pc_ref):
    h = h_ref[...]
    agg = a0_ref[...] + a1_ref[...]
    z = (jnp.dot(h, w1h_ref[...], preferred_element_type=_f32)
         + jnp.dot(agg, w1a_ref[...], preferred_element_type=_f32)
         + b1_ref[...])
    u = jnp.dot(_silu(z), w2_ref[...], preferred_element_type=_f32) + b2_ref[...]
    hn = h + u
    mean = jnp.mean(hn, axis=-1, keepdims=True)
    d = hn - mean
    var = jnp.mean(d * d, axis=-1, keepdims=True)
    hn = d * lax.rsqrt(var + 1e-5) * g_ref[...] + bb_ref[...]
    hn_ref[...] = hn
    pr_ref[...] = jnp.dot(hn, wr_ref[...], preferred_element_type=_f32)
    pc_ref[...] = jnp.dot(hn, wc_ref[...], preferred_element_type=_f32)


def _node_update(h, a0, a1, w1h, w1a, b1, w2, b2, g, bb, wr, wc):
    return pl.pallas_call(
        _node_body,
        grid=(_GN,),
        in_specs=[_rows((_NB, H)), _rows((_NB, H)), _rows((_NB, H)),
                  _full((H, H)), _full((H, H)), _full((1, H)),
                  _full((H, H)), _full((1, H)), _full((1, H)), _full((1, H)),
                  _full((H, ED)), _full((H, ED))],
        out_specs=[_rows((_NB, H)), _rows((_NB, ED)), _rows((_NB, ED))],
        out_shape=[jax.ShapeDtypeStruct((N, H), _f32),
                   jax.ShapeDtypeStruct((N, ED), _f32),
                   jax.ShapeDtypeStruct((N, ED), _f32)],
    )(h, a0, a1, w1h, w1a, b1.reshape(1, H), w2, b2.reshape(1, H),
      g.reshape(1, H), bb.reshape(1, H), wr, wc)


def _edge_body(ga_ref, gb_ref, ea_ref, we_ref, be1_ref, w2_ref, be2_ref, o_ref):
    ea = ea_ref[...]
    z = (ga_ref[...] + gb_ref[...]
         + jnp.dot(ea, we_ref[...], preferred_element_type=_f32) + be1_ref[...])
    o_ref[...] = ea + jnp.dot(_silu(z), w2_ref[...],
                              preferred_element_type=_f32) + be2_ref[...]


def _edge_mlp(ga, gb, ea, we, be1, w2, be2):
    return pl.pallas_call(
        _edge_body,
        grid=(_GE,),
        in_specs=[_rows((_BE, ED)), _rows((_BE, ED)), _rows((_BE, ED)),
                  _full((ED, ED)), _full((1, ED)), _full((ED, ED)), _full((1, ED))],
        out_specs=_rows((_BE, ED)),
        out_shape=jax.ShapeDtypeStruct((E, ED), _f32),
    )(ga, gb, ea, we, be1.reshape(1, ED), w2, be2.reshape(1, ED))


def _readout_body(ea_ref, w1_ref, b1_ref, w2_ref, b2_ref, o_ref):
    z = _silu(jnp.dot(ea_ref[...], w1_ref[...],
                      preferred_element_type=_f32) + b1_ref[...])
    o_ref[...] = jnp.dot(z, w2_ref[...], preferred_element_type=_f32) + b2_ref[...]


def _readout(ea, w1, b1, w2, b2):
    hh = w1.shape[1]
    return pl.pallas_call(
        _readout_body,
        grid=(_GE,),
        in_specs=[_rows((_BE, ED)), _full((ED, hh)), _full((1, hh)),
                  _full((hh, 1)), _full((1, 1))],
        out_specs=_rows((_BE, 1)),
        out_shape=jax.ShapeDtypeStruct((E, 1), _f32),
    )(ea, w1, b1.reshape(1, hh), w2, b2.reshape(1, 1))


def _fc_body(pr_ref, pc_ref, fm_ref, o_ref):
    d = pr_ref[...] - pc_ref[...]
    nrm = jnp.sqrt(jnp.sum(d * d, axis=-1, keepdims=True))
    o_ref[...] = fm_ref[...] * d / (nrm + 1e-8)


def _fc(prow, pcol, fm):
    return pl.pallas_call(
        _fc_body,
        grid=(_GE,),
        in_specs=[_rows((_BE, ED)), _rows((_BE, ED)), _rows((_BE, 1))],
        out_specs=_rows((_BE, ED)),
        out_shape=jax.ShapeDtypeStruct((E, ED), _f32),
    )(prow, pcol, fm)


def _combine_body(p0_ref, p1_ref, o_ref):
    o_ref[...] = (p0_ref[...] + p1_ref[...])[:, :3]


def _combine(p0, p1):
    return pl.pallas_call(
        _combine_body,
        grid=(1,),
        in_specs=[_full((N, ED)), _full((N, ED))],
        out_specs=_full((N, 3)),
        out_shape=jax.ShapeDtypeStruct((N, 3), _f32),
    )(p0, p1)


def kernel(x, pos, edge_index, edge_attr, params):
    row = edge_index[0].astype(jnp.int32)
    col = edge_index[1].astype(jnp.int32)
    row2 = row.reshape(_ECH, _CH)
    col2 = col.reshape(_ECH, _CH)

    h = _embed(x, params['W_ne'], params['b_ne'])

    offs = jnp.linspace(0.0, CUTOFF, ED)
    gamma = -0.5 / (offs[1] - offs[0]) ** 2
    ea = _gauss(edge_attr, offs, gamma)

    layers = params['layers']
    nl = len(layers)
    parts = _agg_sc(h, row2, col2)
    for i, lp in enumerate(layers):
        we1 = lp['We1']
        h, pr, pc = _node_update(
            h, parts[0], parts[1],
            lp['W1'][:H], lp['W1'][H:], lp['b1'], lp['W2'], lp['b2'],
            lp['ln_g'], lp['ln_b'], we1[:H], we1[H:2 * H])
        if i + 1 < nl:
            parts, ga, gb = _aggpair_sc(h, pr, pc, row2, col2)
        else:
            ga, gb = _pair_sc(pr, pc, row2, col2)
        ea = _edge_mlp(ga, gb, ea, we1[2 * H:], lp['be1'], lp['We2'], lp['be2'])

    fm = _readout(ea, params['Wr1'], params['br1'], params['Wr2'], params['br2'])
    posp = jnp.pad(pos, ((0, 0), (0, ED - 3)))
    prow, pcol = _pair_sc(posp, posp, row2, col2)
    fc = _fc(prow, pcol, fm)
    fparts = _scatter16_sc(fc, col2)
    return _combine(fparts[0], fparts[1])


# speedup vs baseline: 2.8192x; 2.8192x over previous
import functools

import jax
import jax.numpy as jnp
from jax import lax
from jax.experimental import pallas as pl
from jax.experimental.pallas import tpu as pltpu
from jax.experimental.pallas import tpu_sc as plsc

N = 10000
E = 320000
H = 128
ED = 16
CUTOFF = 5.0

_NC = 2
_NS = 16
_NW = _NC * _NS
_CH = 128
_ECH = E // _CH
_RPE = _ECH // _NW
_XW = _ECH - _RPE * _NW
_K = 6
_NG = _RPE // _K
_RW = 624
_RREM = N - _NS * _RW
_ROFF = _NS * _RW

_mesh = plsc.VectorSubcoreMesh(core_axis_name="c", subcore_axis_name="s")
_f32 = jnp.float32


def _zero_vmem(ref, nrows, ncols):
    nv = ncols // 16

    def body(i, _):
        for j in range(nv):
            ref[i, pl.ds(j * 16, 16)] = jnp.zeros((16,), _f32)
        return 0

    lax.fori_loop(0, nrows, body, 0)


def _zero_shared_slice(zbuf, acc_sh, sid):
    r0 = pl.multiple_of(sid * _RW, 8)
    nfull = _RW // 128
    rem = _RW - nfull * 128
    for k in range(nfull):
        pltpu.sync_copy(zbuf.at[pl.ds(0, 128)],
                        acc_sh.at[pl.ds(r0 + k * 128, 128)])
    if rem:
        pltpu.sync_copy(zbuf.at[pl.ds(0, rem)],
                        acc_sh.at[pl.ds(r0 + nfull * 128, rem)])

    @pl.when(sid == _NS - 1)
    def _():
        pltpu.sync_copy(zbuf.at[pl.ds(0, _RREM)], acc_sh.at[pl.ds(_ROFF, _RREM)])


def _writeback_slice(acc_sh, out_hbm, cid, sid):
    r0 = pl.multiple_of(sid * _RW, 8)
    pltpu.sync_copy(acc_sh.at[pl.ds(r0, _RW)], out_hbm.at[cid, pl.ds(r0, _RW)])

    @pl.when(sid == _NS - 1)
    def _():
        pltpu.sync_copy(acc_sh.at[pl.ds(_ROFF, _RREM)],
                        out_hbm.at[cid, pl.ds(_ROFF, _RREM)])


_KA = 2
_SB = 26
_NSB = _RPE // _SB
_NGA = _SB // _KA


@functools.partial(
    pl.kernel,
    out_type=jax.ShapeDtypeStruct((_NC, N, H), _f32),
    mesh=_mesh,
    scratch_types=[
        pltpu.VMEM((_SB, _CH), jnp.int32),
        pltpu.VMEM((_SB, _CH), jnp.int32),
        pltpu.VMEM((_KA * _CH, H), _f32),
        pltpu.VMEM_SHARED((N, H), _f32),
        pltpu.SemaphoreType.DMA,
        pltpu.SemaphoreType.DMA,
    ],
    compiler_params=pltpu.CompilerParams(use_tc_tiling_on_sc=False),
)
def _agg_sc(h_hbm, row2_hbm, col2_hbm, out_hbm,
            ridxb, cidxb, rows, agg_sh, gsem, ssem):
    cid = lax.axis_index("c")
    sid = lax.axis_index("s")
    w = cid * _NS + sid
    rb = w * _RPE

    _zero_vmem(rows, 128, H)
    _zero_shared_slice(rows, agg_sh, sid)
    plsc.subcore_barrier()

    def superblock(sb, _):
        pltpu.sync_copy(row2_hbm.at[pl.ds(rb + sb * _SB, _SB)], ridxb)
        pltpu.sync_copy(col2_hbm.at[pl.ds(rb + sb * _SB, _SB)], cidxb)

        def group(g, _):
            gd = [pltpu.async_copy(h_hbm.at[ridxb.at[g * _KA + k]],
                                   rows.at[pl.ds(k * _CH, _CH)], gsem)
                  for k in range(_KA)]
            for d in gd:
                d.wait()
            sd = [pltpu.async_copy(rows.at[pl.ds(k * _CH, _CH)],
                                   agg_sh.at[cidxb.at[g * _KA + k]], ssem,
                                   add=True)
                  for k in range(_KA)]
            for d in sd:
                d.wait()
            return 0

        lax.fori_loop(0, _NGA, group, 0)
        return 0

    lax.fori_loop(0, _NSB, superblock, 0)

    @pl.when(w < _XW)
    def _():
        pltpu.sync_copy(row2_hbm.at[pl.ds(_NW * _RPE + w, 1)],
                        ridxb.at[pl.ds(0, 1)])
        pltpu.sync_copy(col2_hbm.at[pl.ds(_NW * _RPE + w, 1)],
                        cidxb.at[pl.ds(0, 1)])
        pltpu.async_copy(h_hbm.at[ridxb.at[0]],
                         rows.at[pl.ds(0, _CH)], gsem).wait()
        pltpu.async_copy(rows.at[pl.ds(0, _CH)],
                         agg_sh.at[cidxb.at[0]], ssem, add=True).wait()

    plsc.subcore_barrier()
    _writeback_slice(agg_sh, out_hbm, cid, sid)


@functools.partial(
    pl.kernel,
    out_type=[jax.ShapeDtypeStruct((_NC, N, H), _f32),
              jax.ShapeDtypeStruct((E, ED), _f32),
              jax.ShapeDtypeStruct((E, ED), _f32)],
    mesh=_mesh,
    scratch_types=[
        pltpu.VMEM((_SB, _CH), jnp.int32),
        pltpu.VMEM((_SB, _CH), jnp.int32),
        pltpu.VMEM((_KA * _CH, H), _f32),
        pltpu.VMEM((_KA * _CH, ED), _f32),
        pltpu.VMEM((_KA * _CH, ED), _f32),
        pltpu.VMEM_SHARED((N, H), _f32),
        pltpu.SemaphoreType.DMA,
        pltpu.SemaphoreType.DMA,
        pltpu.SemaphoreType.DMA,
    ],
    compiler_params=pltpu.CompilerParams(use_tc_tiling_on_sc=False),
)
def _aggpair_sc(h_hbm, ta_hbm, tb_hbm, row2_hbm, col2_hbm,
                out_hbm, oa_hbm, ob_hbm,
                ridxb, cidxb, rows, pbufa, pbufb, agg_sh, gsem, ssem, wsem):
    cid = lax.axis_index("c")
    sid = lax.axis_index("s")
    w = cid * _NS + sid
    rb = w * _RPE

    _zero_vmem(rows, 128, H)
    _zero_shared_slice(rows, agg_sh, sid)
    plsc.subcore_barrier()

    def superblock(sb, _):
        pltpu.sync_copy(row2_hbm.at[pl.ds(rb + sb * _SB, _SB)], ridxb)
        pltpu.sync_copy(col2_hbm.at[pl.ds(rb + sb * _SB, _SB)], cidxb)

        def group(g, _):
            gd = [pltpu.async_copy(h_hbm.at[ridxb.at[g * _KA + k]],
                                   rows.at[pl.ds(k * _CH, _CH)], gsem)
                  for k in range(_KA)]
            gd += [pltpu.async_copy(ta_hbm.at[ridxb.at[g * _KA + k]],
                                    pbufa.at[pl.ds(k * _CH, _CH)], gsem)
                   for k in range(_KA)]
            gd += [pltpu.async_copy(tb_hbm.at[cidxb.at[g * _KA + k]],
                                    pbufb.at[pl.ds(k * _CH, _CH)], gsem)
                   for k in range(_KA)]
            for d in gd:
                d.wait()
            sd = [pltpu.async_copy(rows.at[pl.ds(k * _CH, _CH)],
                                   agg_sh.at[cidxb.at[g * _KA + k]], ssem,
                                   add=True)
                  for k in range(_KA)]
            e0 = (rb + sb * _SB + g * _KA) * _CH
            wa = pltpu.async_copy(pbufa, oa_hbm.at[pl.ds(e0, _KA * _CH)], wsem)
            wb = pltpu.async_copy(pbufb, ob_hbm.at[pl.ds(e0, _KA * _CH)], wsem)
            for d in sd:
                d.wait()
            wa.wait()
            wb.wait()
            return 0

        lax.fori_loop(0, _NGA, group, 0)
        return 0

    lax.fori_loop(0, _NSB, superblock, 0)

    @pl.when(w < _XW)
    def _():
        pltpu.sync_copy(row2_hbm.at[pl.ds(_NW * _RPE + w, 1)],
                        ridxb.at[pl.ds(0, 1)])
        pltpu.sync_copy(col2_hbm.at[pl.ds(_NW * _RPE + w, 1)],
                        cidxb.at[pl.ds(0, 1)])
        gd = [pltpu.async_copy(h_hbm.at[ridxb.at[0]],
                               rows.at[pl.ds(0, _CH)], gsem),
              pltpu.async_copy(ta_hbm.at[ridxb.at[0]],
                               pbufa.at[pl.ds(0, _CH)], gsem),
              pltpu.async_copy(tb_hbm.at[cidxb.at[0]],
                               pbufb.at[pl.ds(0, _CH)], gsem)]
        for d in gd:
            d.wait()
        e0 = (_NW * _RPE + w) * _CH
        sd = [pltpu.async_copy(rows.at[pl.ds(0, _CH)],
                               agg_sh.at[cidxb.at[0]], ssem, add=True),
              pltpu.async_copy(pbufa.at[pl.ds(0, _CH)],
                               oa_hbm.at[pl.ds(e0, _CH)], wsem),
              pltpu.async_copy(pbufb.at[pl.ds(0, _CH)],
                               ob_hbm.at[pl.ds(e0, _CH)], wsem)]
        for d in sd:
            d.wait()

    plsc.subcore_barrier()
    _writeback_slice(agg_sh, out_hbm, cid, sid)


@functools.partial(
    pl.kernel,
    out_type=[jax.ShapeDtypeStruct((E, ED), _f32),
              jax.ShapeDtypeStruct((E, ED), _f32)],
    mesh=_mesh,
    scratch_types=[
        pltpu.VMEM((_RPE + 1, _CH), jnp.int32),
        pltpu.VMEM((_RPE + 1, _CH), jnp.int32),
        pltpu.VMEM((_K * _CH, ED), _f32),
        pltpu.VMEM((_K * _CH, ED), _f32),
        pltpu.SemaphoreType.DMA,
        pltpu.SemaphoreType.DMA,
    ],
    compiler_params=pltpu.CompilerParams(use_tc_tiling_on_sc=False),
)
def _pair_sc(ta_hbm, tb_hbm, ia2_hbm, ib2_hbm, oa_hbm, ob_hbm,
             idxa2, idxb2, bufa, bufb, gsem, wsem):
    cid = lax.axis_index("c")
    sid = lax.axis_index("s")
    w = cid * _NS + sid
    rb = w * _RPE
    base = rb * _CH

    pltpu.sync_copy(ia2_hbm.at[pl.ds(rb, _RPE)], idxa2.at[pl.ds(0, _RPE)])
    pltpu.sync_copy(ib2_hbm.at[pl.ds(rb, _RPE)], idxb2.at[pl.ds(0, _RPE)])

    @pl.when(w < _XW)
    def _():
        pltpu.sync_copy(ia2_hbm.at[pl.ds(_NW * _RPE + w, 1)],
                        idxa2.at[pl.ds(_RPE, 1)])
        pltpu.sync_copy(ib2_hbm.at[pl.ds(_NW * _RPE + w, 1)],
                        idxb2.at[pl.ds(_RPE, 1)])

    def group(g, _):
        gd = [pltpu.async_copy(ta_hbm.at[idxa2.at[g * _K + k]],
                               bufa.at[pl.ds(k * _CH, _CH)], gsem)
              for k in range(_K)]
        gd += [pltpu.async_copy(tb_hbm.at[idxb2.at[g * _K + k]],
                                bufb.at[pl.ds(k * _CH, _CH)], gsem)
               for k in range(_K)]
        for d in gd:
            d.wait()
        e0 = base + g * _K * _CH
        wa = pltpu.async_copy(bufa, oa_hbm.at[pl.ds(e0, _K * _CH)], wsem)
        wb = pltpu.async_copy(bufb, ob_hbm.at[pl.ds(e0, _K * _CH)], wsem)
        wa.wait()
        wb.wait()
        return 0

    lax.fori_loop(0, _NG, group, 0)

    @pl.when(w < _XW)
    def _():
        ga = pltpu.async_copy(ta_hbm.at[idxa2.at[_RPE]],
                              bufa.at[pl.ds(0, _CH)], gsem)
        gb = pltpu.async_copy(tb_hbm.at[idxb2.at[_RPE]],
                              bufb.at[pl.ds(0, _CH)], gsem)
        ga.wait()
        gb.wait()
        e0 = (_NW * _RPE + w) * _CH
        wa = pltpu.async_copy(bufa.at[pl.ds(0, _CH)],
                              oa_hbm.at[pl.ds(e0, _CH)], wsem)
        wb = pltpu.async_copy(bufb.at[pl.ds(0, _CH)],
                              ob_hbm.at[pl.ds(e0, _CH)], wsem)
        wa.wait()
        wb.wait()


@functools.partial(
    pl.kernel,
    out_type=jax.ShapeDtypeStruct((_NC, N, ED), _f32),
    mesh=_mesh,
    scratch_types=[
        pltpu.VMEM((_RPE + 1, _CH), jnp.int32),
        pltpu.VMEM((_K * _CH, ED), _f32),
        pltpu.VMEM_SHARED((N, ED), _f32),
        pltpu.SemaphoreType.DMA,
        pltpu.SemaphoreType.DMA,
    ],
    compiler_params=pltpu.CompilerParams(use_tc_tiling_on_sc=False),
)
def _scatter16_sc(vals_hbm, col2_hbm, out_hbm, cidx2, vbuf, acc_sh, lsem, ssem):
    cid = lax.axis_index("c")
    sid = lax.axis_index("s")
    w = cid * _NS + sid
    rb = w * _RPE
    base = rb * _CH

    pltpu.sync_copy(col2_hbm.at[pl.ds(rb, _RPE)], cidx2.at[pl.ds(0, _RPE)])

    @pl.when(w < _XW)
    def _():
        pltpu.sync_copy(col2_hbm.at[pl.ds(_NW * _RPE + w, 1)],
                        cidx2.at[pl.ds(_RPE, 1)])

    _zero_vmem(vbuf, 128, ED)
    _zero_shared_slice(vbuf, acc_sh, sid)
    plsc.subcore_barrier()

    def group(g, _):
        e0 = base + g * _K * _CH
        pltpu.async_copy(vals_hbm.at[pl.ds(e0, _K * _CH)], vbuf, lsem).wait()
        sd = [pltpu.async_copy(vbuf.at[pl.ds(k * _CH, _CH)],
                               acc_sh.at[cidx2.at[g * _K + k]], ssem, add=True)
              for k in range(_K)]
        for d in sd:
            d.wait()
        return 0

    lax.fori_loop(0, _NG, group, 0)

    @pl.when(w < _XW)
    def _():
        e0 = (_NW * _RPE + w) * _CH
        pltpu.async_copy(vals_hbm.at[pl.ds(e0, _CH)],
                         vbuf.at[pl.ds(0, _CH)], lsem).wait()
        pltpu.async_copy(vbuf.at[pl.ds(0, _CH)],
                         acc_sh.at[cidx2.at[_RPE]], ssem, add=True).wait()

    plsc.subcore_barrier()
    _writeback_slice(acc_sh, out_hbm, cid, sid)


_NB = 1000
_GN = N // _NB


def _full(shape):
    return pl.BlockSpec(shape, lambda i: tuple(0 for _ in shape))


def _rows(shape):
    return pl.BlockSpec(shape, lambda i: (i,) + tuple(0 for _ in shape[1:]))


def _embed_body(x_ref, w_ref, b_ref, o_ref):
    o_ref[...] = jnp.dot(x_ref[...], w_ref[...],
                         preferred_element_type=_f32) + b_ref[...]


def _embed(x, w, b):
    return pl.pallas_call(
        _embed_body,
        grid=(_GN,),
        in_specs=[_rows((_NB, H)), _full((H, H)), _full((1, H))],
        out_specs=_rows((_NB, H)),
        out_shape=jax.ShapeDtypeStruct((N, H), _f32),
    )(x, w, b.reshape(1, H))


_P = E // 8
_BP = 4000
_GP = _P // _BP


def _gauss_body(d8_ref, r_ref, off_ref, g_ref, o_ref):
    drep = jnp.dot(d8_ref[...], r_ref[...], preferred_element_type=_f32)
    o_ref[...] = jnp.exp(g_ref[0, 0] * (drep - off_ref[...]) ** 2)


def _gauss(edge_attr, rmat, offs_t, gamma):
    return pl.pallas_call(
        _gauss_body,
        grid=(_GP,),
        in_specs=[_rows((_BP, 8)), _full((8, 128)), _full((1, 128)), _full((1, 1))],
        out_specs=_rows((_BP, 128)),
        out_shape=jax.ShapeDtypeStruct((_P, 128), _f32),
    )(edge_attr.reshape(_P, 8), rmat, offs_t.reshape(1, 128),
      gamma.reshape(1, 1))


def _silu(v):
    return v * jax.nn.sigmoid(v)


def _node_body(h_ref, a0_ref, a1_ref, w1h_ref, w1a_ref, b1_ref, w2_ref,
               b2_ref, g_ref, bb_ref, wr_ref, wc_ref,
               hn_ref, pr_ref, pc_ref):
    h = h_ref[...]
    agg = a0_ref[...] + a1_ref[...]
    z = (jnp.dot(h, w1h_ref[...], preferred_element_type=_f32)
         + jnp.dot(agg, w1a_ref[...], preferred_element_type=_f32)
         + b1_ref[...])
    u = jnp.dot(_silu(z), w2_ref[...], preferred_element_type=_f32) + b2_ref[...]
    hn = h + u
    mean = jnp.mean(hn, axis=-1, keepdims=True)
    d = hn - mean
    var = jnp.mean(d * d, axis=-1, keepdims=True)
    hn = d * lax.rsqrt(var + 1e-5) * g_ref[...] + bb_ref[...]
    hn_ref[...] = hn
    pr_ref[...] = jnp.dot(hn, wr_ref[...], preferred_element_type=_f32)
    pc_ref[...] = jnp.dot(hn, wc_ref[...], preferred_element_type=_f32)


def _node_update(h, a0, a1, w1h, w1a, b1, w2, b2, g, bb, wr, wc):
    return pl.pallas_call(
        _node_body,
        grid=(_GN,),
        in_specs=[_rows((_NB, H)), _rows((_NB, H)), _rows((_NB, H)),
                  _full((H, H)), _full((H, H)), _full((1, H)),
                  _full((H, H)), _full((1, H)), _full((1, H)), _full((1, H)),
                  _full((H, ED)), _full((H, ED))],
        out_specs=[_rows((_NB, H)), _rows((_NB, ED)), _rows((_NB, ED))],
        out_shape=[jax.ShapeDtypeStruct((N, H), _f32),
                   jax.ShapeDtypeStruct((N, ED), _f32),
                   jax.ShapeDtypeStruct((N, ED), _f32)],
    )(h, a0, a1, w1h, w1a, b1.reshape(1, H), w2, b2.reshape(1, H),
      g.reshape(1, H), bb.reshape(1, H), wr, wc)


def _edge_body(ga_ref, gb_ref, ea_ref, we_ref, be1_ref, w2_ref, be2_ref, o_ref):
    ea = ea_ref[...]
    z = (ga_ref[...] + gb_ref[...]
         + jnp.dot(ea, we_ref[...], preferred_element_type=_f32) + be1_ref[...])
    o_ref[...] = ea + jnp.dot(_silu(z), w2_ref[...],
                              preferred_element_type=_f32) + be2_ref[...]


def _edge_mlp(ga, gb, ea, kwe, be1t, kw2, be2t):
    return pl.pallas_call(
        _edge_body,
        grid=(_GP,),
        in_specs=[_rows((_BP, 128)), _rows((_BP, 128)), _rows((_BP, 128)),
                  _full((128, 128)), _full((1, 128)), _full((128, 128)),
                  _full((1, 128))],
        out_specs=_rows((_BP, 128)),
        out_shape=jax.ShapeDtypeStruct((_P, 128), _f32),
    )(ga, gb, ea, kwe, be1t.reshape(1, 128), kw2, be2t.reshape(1, 128))


def _force_body(ea_ref, pr_ref, pc_ref, kw1_ref, b1_ref, kw2_ref, b2_ref,
                s_ref, r_ref, o_ref):
    z = _silu(jnp.dot(ea_ref[...], kw1_ref[...],
                      preferred_element_type=_f32) + b1_ref[...])
    fm8 = jnp.dot(z, kw2_ref[...], preferred_element_type=_f32) + b2_ref[0, 0]
    d = pr_ref[...] - pc_ref[...]
    nrm8 = jnp.sqrt(jnp.dot(d * d, s_ref[...], preferred_element_type=_f32))
    scale = jnp.dot(fm8 / (nrm8 + 1e-8), r_ref[...],
                    preferred_element_type=_f32)
    o_ref[...] = scale * d


def _force(ea, prow, pcol, kw1, b1t, kw2, b2, smat, rmat):
    return pl.pallas_call(
        _force_body,
        grid=(_GP,),
        in_specs=[_rows((_BP, 128)), _rows((_BP, 128)), _rows((_BP, 128)),
                  _full((128, 512)), _full((1, 512)), _full((512, 8)),
                  _full((1, 1)), _full((128, 8)), _full((8, 128))],
        out_specs=_rows((_BP, 128)),
        out_shape=jax.ShapeDtypeStruct((_P, 128), _f32),
    )(ea, prow, pcol, kw1, b1t.reshape(1, 512), kw2, b2.reshape(1, 1),
      smat, rmat)


def _combine_body(p0_ref, p1_ref, o_ref):
    o_ref[...] = (p0_ref[...] + p1_ref[...])[:, :3]


def _combine(p0, p1):
    return pl.pallas_call(
        _combine_body,
        grid=(1,),
        in_specs=[_full((N, ED)), _full((N, ED))],
        out_specs=_full((N, 3)),
        out_shape=jax.ShapeDtypeStruct((N, 3), _f32),
    )(p0, p1)


def kernel(x, pos, edge_index, edge_attr, params):
    row = edge_index[0].astype(jnp.int32)
    col = edge_index[1].astype(jnp.int32)
    row2 = row.reshape(_ECH, _CH)
    col2 = col.reshape(_ECH, _CH)

    h = _embed(x, params['W_ne'], params['b_ne'])

    eye8 = jnp.eye(8, dtype=_f32)
    rmat = jnp.kron(eye8, jnp.ones((1, ED), _f32))
    smat = jnp.kron(eye8, jnp.ones((ED, 1), _f32))
    offs = jnp.linspace(0.0, CUTOFF, ED)
    gamma = -0.5 / (offs[1] - offs[0]) ** 2
    ea = _gauss(edge_attr, rmat, jnp.tile(offs, 8), gamma)

    layers = params['layers']
    nl = len(layers)
    parts = _agg_sc(h, row2, col2)
    for i, lp in enumerate(layers):
        we1 = lp['We1']
        h, pr, pc = _node_update(
            h, parts[0], parts[1],
            lp['W1'][:H], lp['W1'][H:], lp['b1'], lp['W2'], lp['b2'],
            lp['ln_g'], lp['ln_b'], we1[:H], we1[H:2 * H])
        if i + 1 < nl:
            parts, ga, gb = _aggpair_sc(h, pr, pc, row2, col2)
        else:
            ga, gb = _pair_sc(pr, pc, row2, col2)
        ea = _edge_mlp(ga.reshape(_P, 128), gb.reshape(_P, 128), ea,
                       jnp.kron(eye8, we1[2 * H:]), jnp.tile(lp['be1'], 8),
                       jnp.kron(eye8, lp['We2']), jnp.tile(lp['be2'], 8))

    posp = jnp.pad(pos, ((0, 0), (0, ED - 3)))
    prow, pcol = _pair_sc(posp, posp, row2, col2)
    fc = _force(ea, prow.reshape(_P, 128), pcol.reshape(_P, 128),
                jnp.kron(eye8, params['Wr1']), jnp.tile(params['br1'], 8),
                jnp.kron(eye8, params['Wr2']), params['br2'], smat, rmat)
    fparts = _scatter16_sc(fc.reshape(E, ED), col2)
    return _combine(fparts[0], fparts[1])


# speedup vs baseline: 2.9173x; 1.0348x over previous
import functools

import jax
import jax.numpy as jnp
from jax import lax
from jax.experimental import pallas as pl
from jax.experimental.pallas import tpu as pltpu
from jax.experimental.pallas import tpu_sc as plsc

N = 10000
E = 320000
H = 128
ED = 16
CUTOFF = 5.0

_NC = 2
_NS = 16
_NW = _NC * _NS
_CH = 128
_ECH = E // _CH
_RPE = _ECH // _NW
_XW = _ECH - _RPE * _NW
_K = 6
_NG = _RPE // _K
_RW = 624
_RREM = N - _NS * _RW
_ROFF = _NS * _RW

_mesh = plsc.VectorSubcoreMesh(core_axis_name="c", subcore_axis_name="s")
_f32 = jnp.float32


def _zero_vmem(ref, nrows, ncols):
    nv = ncols // 16

    def body(i, _):
        for j in range(nv):
            ref[i, pl.ds(j * 16, 16)] = jnp.zeros((16,), _f32)
        return 0

    lax.fori_loop(0, nrows, body, 0)


def _zero_shared_slice(zbuf, acc_sh, sid):
    r0 = pl.multiple_of(sid * _RW, 8)
    nfull = _RW // 128
    rem = _RW - nfull * 128
    for k in range(nfull):
        pltpu.sync_copy(zbuf.at[pl.ds(0, 128)],
                        acc_sh.at[pl.ds(r0 + k * 128, 128)])
    if rem:
        pltpu.sync_copy(zbuf.at[pl.ds(0, rem)],
                        acc_sh.at[pl.ds(r0 + nfull * 128, rem)])

    @pl.when(sid == _NS - 1)
    def _():
        pltpu.sync_copy(zbuf.at[pl.ds(0, _RREM)], acc_sh.at[pl.ds(_ROFF, _RREM)])


def _writeback_slice(acc_sh, out_hbm, cid, sid):
    r0 = pl.multiple_of(sid * _RW, 8)
    pltpu.sync_copy(acc_sh.at[pl.ds(r0, _RW)], out_hbm.at[cid, pl.ds(r0, _RW)])

    @pl.when(sid == _NS - 1)
    def _():
        pltpu.sync_copy(acc_sh.at[pl.ds(_ROFF, _RREM)],
                        out_hbm.at[cid, pl.ds(_ROFF, _RREM)])


_KA = 1
_SB = 13
_NSB = _RPE // _SB
_NGA = _SB // _KA


def _drain(src_hbm, dst_vmem, sem, n):
    for _ in range(n):
        pltpu.make_async_copy(src_hbm, dst_vmem, sem).wait()


@functools.partial(
    pl.kernel,
    out_type=jax.ShapeDtypeStruct((_NC, N, H), _f32),
    mesh=_mesh,
    scratch_types=[
        pltpu.VMEM((_SB, _CH), jnp.int32),
        pltpu.VMEM((_SB, _CH), jnp.int32),
        pltpu.VMEM((2 * _KA * _CH, H), _f32),
        pltpu.VMEM_SHARED((N, H), _f32),
        pltpu.SemaphoreType.DMA,
        pltpu.SemaphoreType.DMA,
    ],
    compiler_params=pltpu.CompilerParams(use_tc_tiling_on_sc=False),
)
def _agg_sc(h_hbm, row2_hbm, col2_hbm, out_hbm,
            ridxb, cidxb, rows, agg_sh, gsem, ssem):
    cid = lax.axis_index("c")
    sid = lax.axis_index("s")
    w = cid * _NS + sid
    rb = w * _RPE

    _zero_vmem(rows, 128, H)
    _zero_shared_slice(rows, agg_sh, sid)
    plsc.subcore_barrier()

    def fire_g(g, s):
        for k in range(_KA):
            pltpu.async_copy(h_hbm.at[ridxb.at[g * _KA + k]],
                             rows.at[pl.ds((s * _KA + k) * _CH, _CH)], gsem)

    def superblock(sb, _):
        pltpu.sync_copy(row2_hbm.at[pl.ds(rb + sb * _SB, _SB)], ridxb)
        pltpu.sync_copy(col2_hbm.at[pl.ds(rb + sb * _SB, _SB)], cidxb)
        fire_g(0, 0)

        def group(g, _):
            s = g % 2
            _drain(h_hbm.at[pl.ds(0, _CH)], rows.at[pl.ds(0, _CH)], gsem, _KA)

            @pl.when(g >= 1)
            def _():
                _drain(h_hbm.at[pl.ds(0, _CH)], rows.at[pl.ds(0, _CH)],
                       ssem, _KA)

            @pl.when(g + 1 < _NGA)
            def _():
                fire_g(g + 1, 1 - s)

            for k in range(_KA):
                pltpu.async_copy(rows.at[pl.ds((s * _KA + k) * _CH, _CH)],
                                 agg_sh.at[cidxb.at[g * _KA + k]], ssem,
                                 add=True)
            return 0

        lax.fori_loop(0, _NGA, group, 0)
        _drain(h_hbm.at[pl.ds(0, _CH)], rows.at[pl.ds(0, _CH)], ssem, _KA)
        return 0

    lax.fori_loop(0, _NSB, superblock, 0)

    @pl.when(w < _XW)
    def _():
        pltpu.sync_copy(row2_hbm.at[pl.ds(_NW * _RPE + w, 1)],
                        ridxb.at[pl.ds(0, 1)])
        pltpu.sync_copy(col2_hbm.at[pl.ds(_NW * _RPE + w, 1)],
                        cidxb.at[pl.ds(0, 1)])
        pltpu.async_copy(h_hbm.at[ridxb.at[0]],
                         rows.at[pl.ds(0, _CH)], gsem).wait()
        pltpu.async_copy(rows.at[pl.ds(0, _CH)],
                         agg_sh.at[cidxb.at[0]], ssem, add=True).wait()

    plsc.subcore_barrier()
    _writeback_slice(agg_sh, out_hbm, cid, sid)


@functools.partial(
    pl.kernel,
    out_type=[jax.ShapeDtypeStruct((_NC, N, H), _f32),
              jax.ShapeDtypeStruct((E, ED), _f32),
              jax.ShapeDtypeStruct((E, ED), _f32)],
    mesh=_mesh,
    scratch_types=[
        pltpu.VMEM((_SB, _CH), jnp.int32),
        pltpu.VMEM((_SB, _CH), jnp.int32),
        pltpu.VMEM((2 * _KA * _CH, H), _f32),
        pltpu.VMEM((2 * _KA * _CH, ED), _f32),
        pltpu.VMEM((2 * _KA * _CH, ED), _f32),
        pltpu.VMEM_SHARED((N, H), _f32),
        pltpu.SemaphoreType.DMA,
        pltpu.SemaphoreType.DMA,
        pltpu.SemaphoreType.DMA,
    ],
    compiler_params=pltpu.CompilerParams(use_tc_tiling_on_sc=False),
)
def _aggpair_sc(h_hbm, ta_hbm, tb_hbm, row2_hbm, col2_hbm,
                out_hbm, oa_hbm, ob_hbm,
                ridxb, cidxb, rows, pbufa, pbufb, agg_sh, gsem, ssem, wsem):
    cid = lax.axis_index("c")
    sid = lax.axis_index("s")
    w = cid * _NS + sid
    rb = w * _RPE

    _zero_vmem(rows, 128, H)
    _zero_shared_slice(rows, agg_sh, sid)
    plsc.subcore_barrier()

    def fire_g(g, s):
        for k in range(_KA):
            pltpu.async_copy(h_hbm.at[ridxb.at[g * _KA + k]],
                             rows.at[pl.ds((s * _KA + k) * _CH, _CH)], gsem)
            pltpu.async_copy(ta_hbm.at[ridxb.at[g * _KA + k]],
                             pbufa.at[pl.ds((s * _KA + k) * _CH, _CH)], gsem)
            pltpu.async_copy(tb_hbm.at[cidxb.at[g * _KA + k]],
                             pbufb.at[pl.ds((s * _KA + k) * _CH, _CH)], gsem)

    def drain_sw(n):
        _drain(h_hbm.at[pl.ds(0, _CH)], rows.at[pl.ds(0, _CH)], ssem, n)
        _drain(oa_hbm.at[pl.ds(0, _KA * _CH)],
               pbufa.at[pl.ds(0, _KA * _CH)], wsem, 2)

    def superblock(sb, _):
        pltpu.sync_copy(row2_hbm.at[pl.ds(rb + sb * _SB, _SB)], ridxb)
        pltpu.sync_copy(col2_hbm.at[pl.ds(rb + sb * _SB, _SB)], cidxb)
        fire_g(0, 0)

        def group(g, _):
            s = g % 2
            _drain(h_hbm.at[pl.ds(0, _CH)], rows.at[pl.ds(0, _CH)], gsem, _KA)
            _drain(ta_hbm.at[pl.ds(0, _CH)], pbufa.at[pl.ds(0, _CH)],
                   gsem, 2 * _KA)

            @pl.when(g >= 1)
            def _():
                drain_sw(_KA)

            @pl.when(g + 1 < _NGA)
            def _():
                fire_g(g + 1, 1 - s)

            for k in range(_KA):
                pltpu.async_copy(rows.at[pl.ds((s * _KA + k) * _CH, _CH)],
                                 agg_sh.at[cidxb.at[g * _KA + k]], ssem,
                                 add=True)
            e0 = (rb + sb * _SB + g * _KA) * _CH
            pltpu.async_copy(pbufa.at[pl.ds(s * _KA * _CH, _KA * _CH)],
                             oa_hbm.at[pl.ds(e0, _KA * _CH)], wsem)
            pltpu.async_copy(pbufb.at[pl.ds(s * _KA * _CH, _KA * _CH)],
                             ob_hbm.at[pl.ds(e0, _KA * _CH)], wsem)
            return 0

        lax.fori_loop(0, _NGA, group, 0)
        drain_sw(_KA)
        return 0

    lax.fori_loop(0, _NSB, superblock, 0)

    @pl.when(w < _XW)
    def _():
        pltpu.sync_copy(row2_hbm.at[pl.ds(_NW * _RPE + w, 1)],
                        ridxb.at[pl.ds(0, 1)])
        pltpu.sync_copy(col2_hbm.at[pl.ds(_NW * _RPE + w, 1)],
                        cidxb.at[pl.ds(0, 1)])
        gd = [pltpu.async_copy(h_hbm.at[ridxb.at[0]],
                               rows.at[pl.ds(0, _CH)], gsem),
              pltpu.async_copy(ta_hbm.at[ridxb.at[0]],
                               pbufa.at[pl.ds(0, _CH)], gsem),
              pltpu.async_copy(tb_hbm.at[cidxb.at[0]],
                               pbufb.at[pl.ds(0, _CH)], gsem)]
        for d in gd:
            d.wait()
        e0 = (_NW * _RPE + w) * _CH
        sd = [pltpu.async_copy(rows.at[pl.ds(0, _CH)],
                               agg_sh.at[cidxb.at[0]], ssem, add=True),
              pltpu.async_copy(pbufa.at[pl.ds(0, _CH)],
                               oa_hbm.at[pl.ds(e0, _CH)], wsem),
              pltpu.async_copy(pbufb.at[pl.ds(0, _CH)],
                               ob_hbm.at[pl.ds(e0, _CH)], wsem)]
        for d in sd:
            d.wait()

    plsc.subcore_barrier()
    _writeback_slice(agg_sh, out_hbm, cid, sid)


@functools.partial(
    pl.kernel,
    out_type=[jax.ShapeDtypeStruct((E, ED), _f32),
              jax.ShapeDtypeStruct((E, ED), _f32)],
    mesh=_mesh,
    scratch_types=[
        pltpu.VMEM((_RPE + 1, _CH), jnp.int32),
        pltpu.VMEM((_RPE + 1, _CH), jnp.int32),
        pltpu.VMEM((_K * _CH, ED), _f32),
        pltpu.VMEM((_K * _CH, ED), _f32),
        pltpu.SemaphoreType.DMA,
        pltpu.SemaphoreType.DMA,
    ],
    compiler_params=pltpu.CompilerParams(use_tc_tiling_on_sc=False),
)
def _pair_sc(ta_hbm, tb_hbm, ia2_hbm, ib2_hbm, oa_hbm, ob_hbm,
             idxa2, idxb2, bufa, bufb, gsem, wsem):
    cid = lax.axis_index("c")
    sid = lax.axis_index("s")
    w = cid * _NS + sid
    rb = w * _RPE
    base = rb * _CH

    pltpu.sync_copy(ia2_hbm.at[pl.ds(rb, _RPE)], idxa2.at[pl.ds(0, _RPE)])
    pltpu.sync_copy(ib2_hbm.at[pl.ds(rb, _RPE)], idxb2.at[pl.ds(0, _RPE)])

    @pl.when(w < _XW)
    def _():
        pltpu.sync_copy(ia2_hbm.at[pl.ds(_NW * _RPE + w, 1)],
                        idxa2.at[pl.ds(_RPE, 1)])
        pltpu.sync_copy(ib2_hbm.at[pl.ds(_NW * _RPE + w, 1)],
                        idxb2.at[pl.ds(_RPE, 1)])

    def group(g, _):
        gd = [pltpu.async_copy(ta_hbm.at[idxa2.at[g * _K + k]],
                               bufa.at[pl.ds(k * _CH, _CH)], gsem)
              for k in range(_K)]
        gd += [pltpu.async_copy(tb_hbm.at[idxb2.at[g * _K + k]],
                                bufb.at[pl.ds(k * _CH, _CH)], gsem)
               for k in range(_K)]
        for d in gd:
            d.wait()
        e0 = base + g * _K * _CH
        wa = pltpu.async_copy(bufa, oa_hbm.at[pl.ds(e0, _K * _CH)], wsem)
        wb = pltpu.async_copy(bufb, ob_hbm.at[pl.ds(e0, _K * _CH)], wsem)
        wa.wait()
        wb.wait()
        return 0

    lax.fori_loop(0, _NG, group, 0)

    @pl.when(w < _XW)
    def _():
        ga = pltpu.async_copy(ta_hbm.at[idxa2.at[_RPE]],
                              bufa.at[pl.ds(0, _CH)], gsem)
        gb = pltpu.async_copy(tb_hbm.at[idxb2.at[_RPE]],
                              bufb.at[pl.ds(0, _CH)], gsem)
        ga.wait()
        gb.wait()
        e0 = (_NW * _RPE + w) * _CH
        wa = pltpu.async_copy(bufa.at[pl.ds(0, _CH)],
                              oa_hbm.at[pl.ds(e0, _CH)], wsem)
        wb = pltpu.async_copy(bufb.at[pl.ds(0, _CH)],
                              ob_hbm.at[pl.ds(e0, _CH)], wsem)
        wa.wait()
        wb.wait()


@functools.partial(
    pl.kernel,
    out_type=jax.ShapeDtypeStruct((_NC, N, ED), _f32),
    mesh=_mesh,
    scratch_types=[
        pltpu.VMEM((_RPE + 1, _CH), jnp.int32),
        pltpu.VMEM((_K * _CH, ED), _f32),
        pltpu.VMEM_SHARED((N, ED), _f32),
        pltpu.SemaphoreType.DMA,
        pltpu.SemaphoreType.DMA,
    ],
    compiler_params=pltpu.CompilerParams(use_tc_tiling_on_sc=False),
)
def _scatter16_sc(vals_hbm, col2_hbm, out_hbm, cidx2, vbuf, acc_sh, lsem, ssem):
    cid = lax.axis_index("c")
    sid = lax.axis_index("s")
    w = cid * _NS + sid
    rb = w * _RPE
    base = rb * _CH

    pltpu.sync_copy(col2_hbm.at[pl.ds(rb, _RPE)], cidx2.at[pl.ds(0, _RPE)])

    @pl.when(w < _XW)
    def _():
        pltpu.sync_copy(col2_hbm.at[pl.ds(_NW * _RPE + w, 1)],
                        cidx2.at[pl.ds(_RPE, 1)])

    _zero_vmem(vbuf, 128, ED)
    _zero_shared_slice(vbuf, acc_sh, sid)
    plsc.subcore_barrier()

    def group(g, _):
        e0 = base + g * _K * _CH
        pltpu.async_copy(vals_hbm.at[pl.ds(e0, _K * _CH)], vbuf, lsem).wait()
        sd = [pltpu.async_copy(vbuf.at[pl.ds(k * _CH, _CH)],
                               acc_sh.at[cidx2.at[g * _K + k]], ssem, add=True)
              for k in range(_K)]
        for d in sd:
            d.wait()
        return 0

    lax.fori_loop(0, _NG, group, 0)

    @pl.when(w < _XW)
    def _():
        e0 = (_NW * _RPE + w) * _CH
        pltpu.async_copy(vals_hbm.at[pl.ds(e0, _CH)],
                         vbuf.at[pl.ds(0, _CH)], lsem).wait()
        pltpu.async_copy(vbuf.at[pl.ds(0, _CH)],
                         acc_sh.at[cidx2.at[_RPE]], ssem, add=True).wait()

    plsc.subcore_barrier()
    _writeback_slice(acc_sh, out_hbm, cid, sid)


_NB = 1000
_GN = N // _NB


def _full(shape):
    return pl.BlockSpec(shape, lambda i: tuple(0 for _ in shape))


def _rows(shape):
    return pl.BlockSpec(shape, lambda i: (i,) + tuple(0 for _ in shape[1:]))


def _embed_body(x_ref, w_ref, b_ref, o_ref):
    o_ref[...] = jnp.dot(x_ref[...], w_ref[...],
                         preferred_element_type=_f32) + b_ref[...]


def _embed(x, w, b):
    return pl.pallas_call(
        _embed_body,
        grid=(_GN,),
        in_specs=[_rows((_NB, H)), _full((H, H)), _full((1, H))],
        out_specs=_rows((_NB, H)),
        out_shape=jax.ShapeDtypeStruct((N, H), _f32),
    )(x, w, b.reshape(1, H))


_P = E // 8
_BP = 4000
_GP = _P // _BP


def _gauss_body(d8_ref, r_ref, off_ref, g_ref, o_ref):
    drep = jnp.dot(d8_ref[...], r_ref[...], preferred_element_type=_f32)
    o_ref[...] = jnp.exp(g_ref[0, 0] * (drep - off_ref[...]) ** 2)


def _gauss(edge_attr, rmat, offs_t, gamma):
    return pl.pallas_call(
        _gauss_body,
        grid=(_GP,),
        in_specs=[_rows((_BP, 8)), _full((8, 128)), _full((1, 128)), _full((1, 1))],
        out_specs=_rows((_BP, 128)),
        out_shape=jax.ShapeDtypeStruct((_P, 128), _f32),
    )(edge_attr.reshape(_P, 8), rmat, offs_t.reshape(1, 128),
      gamma.reshape(1, 1))


def _silu(v):
    return v * jax.nn.sigmoid(v)


def _node_body(h_ref, a0_ref, a1_ref, w1h_ref, w1a_ref, b1_ref, w2_ref,
               b2_ref, g_ref, bb_ref, wr_ref, wc_ref,
               hn_ref, pr_ref, pc_ref):
    h = h_ref[...]
    agg = a0_ref[...] + a1_ref[...]
    z = (jnp.dot(h, w1h_ref[...], preferred_element_type=_f32)
         + jnp.dot(agg, w1a_ref[...], preferred_element_type=_f32)
         + b1_ref[...])
    u = jnp.dot(_silu(z), w2_ref[...], preferred_element_type=_f32) + b2_ref[...]
    hn = h + u
    mean = jnp.mean(hn, axis=-1, keepdims=True)
    d = hn - mean
    var = jnp.mean(d * d, axis=-1, keepdims=True)
    hn = d * lax.rsqrt(var + 1e-5) * g_ref[...] + bb_ref[...]
    hn_ref[...] = hn
    pr_ref[...] = jnp.dot(hn, wr_ref[...], preferred_element_type=_f32)
    pc_ref[...] = jnp.dot(hn, wc_ref[...], preferred_element_type=_f32)


def _node_update(h, a0, a1, w1h, w1a, b1, w2, b2, g, bb, wr, wc):
    return pl.pallas_call(
        _node_body,
        grid=(_GN,),
        in_specs=[_rows((_NB, H)), _rows((_NB, H)), _rows((_NB, H)),
                  _full((H, H)), _full((H, H)), _full((1, H)),
                  _full((H, H)), _full((1, H)), _full((1, H)), _full((1, H)),
                  _full((H, ED)), _full((H, ED))],
        out_specs=[_rows((_NB, H)), _rows((_NB, ED)), _rows((_NB, ED))],
        out_shape=[jax.ShapeDtypeStruct((N, H), _f32),
                   jax.ShapeDtypeStruct((N, ED), _f32),
                   jax.ShapeDtypeStruct((N, ED), _f32)],
    )(h, a0, a1, w1h, w1a, b1.reshape(1, H), w2, b2.reshape(1, H),
      g.reshape(1, H), bb.reshape(1, H), wr, wc)


def _edge_body(ga_ref, gb_ref, ea_ref, we_ref, be1_ref, w2_ref, be2_ref, o_ref):
    ea = ea_ref[...]
    z = (ga_ref[...] + gb_ref[...]
         + jnp.dot(ea, we_ref[...], preferred_element_type=_f32) + be1_ref[...])
    o_ref[...] = ea + jnp.dot(_silu(z), w2_ref[...],
                              preferred_element_type=_f32) + be2_ref[...]


def _edge_mlp(ga, gb, ea, kwe, be1t, kw2, be2t):
    return pl.pallas_call(
        _edge_body,
        grid=(_GP,),
        in_specs=[_rows((_BP, 128)), _rows((_BP, 128)), _rows((_BP, 128)),
                  _full((128, 128)), _full((1, 128)), _full((128, 128)),
                  _full((1, 128))],
        out_specs=_rows((_BP, 128)),
        out_shape=jax.ShapeDtypeStruct((_P, 128), _f32),
    )(ga, gb, ea, kwe, be1t.reshape(1, 128), kw2, be2t.reshape(1, 128))


def _force_body(ea_ref, pr_ref, pc_ref, kw1_ref, b1_ref, kw2_ref, b2_ref,
                s_ref, r_ref, o_ref):
    z = _silu(jnp.dot(ea_ref[...], kw1_ref[...],
                      preferred_element_type=_f32) + b1_ref[...])
    fm8 = jnp.dot(z, kw2_ref[...], preferred_element_type=_f32) + b2_ref[0, 0]
    d = pr_ref[...] - pc_ref[...]
    nrm8 = jnp.sqrt(jnp.dot(d * d, s_ref[...], preferred_element_type=_f32))
    scale = jnp.dot(fm8 / (nrm8 + 1e-8), r_ref[...],
                    preferred_element_type=_f32)
    o_ref[...] = scale * d


def _force(ea, prow, pcol, kw1, b1t, kw2, b2, smat, rmat):
    return pl.pallas_call(
        _force_body,
        grid=(_GP,),
        in_specs=[_rows((_BP, 128)), _rows((_BP, 128)), _rows((_BP, 128)),
                  _full((128, 512)), _full((1, 512)), _full((512, 8)),
                  _full((1, 1)), _full((128, 8)), _full((8, 128))],
        out_specs=_rows((_BP, 128)),
        out_shape=jax.ShapeDtypeStruct((_P, 128), _f32),
    )(ea, prow, pcol, kw1, b1t.reshape(1, 512), kw2, b2.reshape(1, 1),
      smat, rmat)


def _combine_body(p0_ref, p1_ref, o_ref):
    o_ref[...] = (p0_ref[...] + p1_ref[...])[:, :3]


def _combine(p0, p1):
    return pl.pallas_call(
        _combine_body,
        grid=(1,),
        in_specs=[_full((N, ED)), _full((N, ED))],
        out_specs=_full((N, 3)),
        out_shape=jax.ShapeDtypeStruct((N, 3), _f32),
    )(p0, p1)


def kernel(x, pos, edge_index, edge_attr, params):
    row = edge_index[0].astype(jnp.int32)
    col = edge_index[1].astype(jnp.int32)
    row2 = row.reshape(_ECH, _CH)
    col2 = col.reshape(_ECH, _CH)

    h = _embed(x, params['W_ne'], params['b_ne'])

    eye8 = jnp.eye(8, dtype=_f32)
    rmat = jnp.kron(eye8, jnp.ones((1, ED), _f32))
    smat = jnp.kron(eye8, jnp.ones((ED, 1), _f32))
    offs = jnp.linspace(0.0, CUTOFF, ED)
    gamma = -0.5 / (offs[1] - offs[0]) ** 2
    ea = _gauss(edge_attr, rmat, jnp.tile(offs, 8), gamma)

    layers = params['layers']
    nl = len(layers)
    parts = _agg_sc(h, row2, col2)
    for i, lp in enumerate(layers):
        we1 = lp['We1']
        h, pr, pc = _node_update(
            h, parts[0], parts[1],
            lp['W1'][:H], lp['W1'][H:], lp['b1'], lp['W2'], lp['b2'],
            lp['ln_g'], lp['ln_b'], we1[:H], we1[H:2 * H])
        if i + 1 < nl:
            parts, ga, gb = _aggpair_sc(h, pr, pc, row2, col2)
        else:
            ga, gb = _pair_sc(pr, pc, row2, col2)
        ea = _edge_mlp(ga.reshape(_P, 128), gb.reshape(_P, 128), ea,
                       jnp.kron(eye8, we1[2 * H:]), jnp.tile(lp['be1'], 8),
                       jnp.kron(eye8, lp['We2']), jnp.tile(lp['be2'], 8))

    posp = jnp.pad(pos, ((0, 0), (0, ED - 3)))
    prow, pcol = _pair_sc(posp, posp, row2, col2)
    fc = _force(ea, prow.reshape(_P, 128), pcol.reshape(_P, 128),
                jnp.kron(eye8, params['Wr1']), jnp.tile(params['br1'], 8),
                jnp.kron(eye8, params['Wr2']), params['br2'], smat, rmat)
    fparts = _scatter16_sc(fc.reshape(E, ED), col2)
    return _combine(fparts[0], fparts[1])


# speedup vs baseline: 3.0053x; 1.0302x over previous
import functools

import jax
import jax.numpy as jnp
from jax import lax
from jax.experimental import pallas as pl
from jax.experimental.pallas import tpu as pltpu
from jax.experimental.pallas import tpu_sc as plsc

N = 10000
E = 320000
H = 128
ED = 16
CUTOFF = 5.0

_NC = 2
_NS = 16
_NW = _NC * _NS
_CH = 128
_ECH = E // _CH
_RPE = _ECH // _NW
_XW = _ECH - _RPE * _NW
_K = 6
_NG = _RPE // _K
_RW = 624
_RREM = N - _NS * _RW
_ROFF = _NS * _RW

_mesh = plsc.VectorSubcoreMesh(core_axis_name="c", subcore_axis_name="s")
_f32 = jnp.float32


def _zero_vmem(ref, nrows, ncols):
    nv = ncols // 16

    def body(i, _):
        for j in range(nv):
            ref[i, pl.ds(j * 16, 16)] = jnp.zeros((16,), _f32)
        return 0

    lax.fori_loop(0, nrows, body, 0)


def _zero_shared_slice(zbuf, acc_sh, sid):
    r0 = pl.multiple_of(sid * _RW, 8)
    nfull = _RW // 128
    rem = _RW - nfull * 128
    for k in range(nfull):
        pltpu.sync_copy(zbuf.at[pl.ds(0, 128)],
                        acc_sh.at[pl.ds(r0 + k * 128, 128)])
    if rem:
        pltpu.sync_copy(zbuf.at[pl.ds(0, rem)],
                        acc_sh.at[pl.ds(r0 + nfull * 128, rem)])

    @pl.when(sid == _NS - 1)
    def _():
        pltpu.sync_copy(zbuf.at[pl.ds(0, _RREM)], acc_sh.at[pl.ds(_ROFF, _RREM)])


def _writeback_slice(acc_sh, out_hbm, cid, sid):
    r0 = pl.multiple_of(sid * _RW, 8)
    pltpu.sync_copy(acc_sh.at[pl.ds(r0, _RW)], out_hbm.at[cid, pl.ds(r0, _RW)])

    @pl.when(sid == _NS - 1)
    def _():
        pltpu.sync_copy(acc_sh.at[pl.ds(_ROFF, _RREM)],
                        out_hbm.at[cid, pl.ds(_ROFF, _RREM)])


_KA = 1
_SB = 13
_NSB = _RPE // _SB
_NGA = _SB // _KA


def _drain(src_hbm, dst_vmem, sem, n):
    for _ in range(n):
        pltpu.make_async_copy(src_hbm, dst_vmem, sem).wait()


@functools.partial(
    pl.kernel,
    out_type=jax.ShapeDtypeStruct((_NC, N, H), _f32),
    mesh=_mesh,
    scratch_types=[
        pltpu.VMEM((_SB, _CH), jnp.int32),
        pltpu.VMEM((_SB, _CH), jnp.int32),
        pltpu.VMEM((2 * _KA * _CH, H), _f32),
        pltpu.VMEM_SHARED((N, H), _f32),
        pltpu.SemaphoreType.DMA,
        pltpu.SemaphoreType.DMA,
    ],
    compiler_params=pltpu.CompilerParams(use_tc_tiling_on_sc=False),
)
def _agg_sc(h_hbm, row2_hbm, col2_hbm, out_hbm,
            ridxb, cidxb, rows, agg_sh, gsem, ssem):
    cid = lax.axis_index("c")
    sid = lax.axis_index("s")
    w = cid * _NS + sid
    rb = w * _RPE

    _zero_vmem(rows, 128, H)
    _zero_shared_slice(rows, agg_sh, sid)
    plsc.subcore_barrier()

    def fire_g(g, s):
        for k in range(_KA):
            pltpu.async_copy(h_hbm.at[ridxb.at[g * _KA + k]],
                             rows.at[pl.ds((s * _KA + k) * _CH, _CH)], gsem)

    def superblock(sb, _):
        pltpu.sync_copy(row2_hbm.at[pl.ds(rb + sb * _SB, _SB)], ridxb)
        pltpu.sync_copy(col2_hbm.at[pl.ds(rb + sb * _SB, _SB)], cidxb)
        fire_g(0, 0)

        def group(g, _):
            s = g % 2
            _drain(h_hbm.at[pl.ds(0, _CH)], rows.at[pl.ds(0, _CH)], gsem, _KA)

            @pl.when(g >= 1)
            def _():
                _drain(h_hbm.at[pl.ds(0, _CH)], rows.at[pl.ds(0, _CH)],
                       ssem, _KA)

            @pl.when(g + 1 < _NGA)
            def _():
                fire_g(g + 1, 1 - s)

            for k in range(_KA):
                pltpu.async_copy(rows.at[pl.ds((s * _KA + k) * _CH, _CH)],
                                 agg_sh.at[cidxb.at[g * _KA + k]], ssem,
                                 add=True)
            return 0

        lax.fori_loop(0, _NGA, group, 0)
        _drain(h_hbm.at[pl.ds(0, _CH)], rows.at[pl.ds(0, _CH)], ssem, _KA)
        return 0

    lax.fori_loop(0, _NSB, superblock, 0)

    @pl.when(w < _XW)
    def _():
        pltpu.sync_copy(row2_hbm.at[pl.ds(_NW * _RPE + w, 1)],
                        ridxb.at[pl.ds(0, 1)])
        pltpu.sync_copy(col2_hbm.at[pl.ds(_NW * _RPE + w, 1)],
                        cidxb.at[pl.ds(0, 1)])
        pltpu.async_copy(h_hbm.at[ridxb.at[0]],
                         rows.at[pl.ds(0, _CH)], gsem).wait()
        pltpu.async_copy(rows.at[pl.ds(0, _CH)],
                         agg_sh.at[cidxb.at[0]], ssem, add=True).wait()

    plsc.subcore_barrier()
    _writeback_slice(agg_sh, out_hbm, cid, sid)


@functools.partial(
    pl.kernel,
    out_type=[jax.ShapeDtypeStruct((_NC, N, H), _f32),
              jax.ShapeDtypeStruct((E, ED), _f32),
              jax.ShapeDtypeStruct((E, ED), _f32)],
    mesh=_mesh,
    scratch_types=[
        pltpu.VMEM((_SB, _CH), jnp.int32),
        pltpu.VMEM((_SB, _CH), jnp.int32),
        pltpu.VMEM((2 * _KA * _CH, H), _f32),
        pltpu.VMEM((2 * _KA * _CH, ED), _f32),
        pltpu.VMEM((2 * _KA * _CH, ED), _f32),
        pltpu.VMEM_SHARED((N, H), _f32),
        pltpu.SemaphoreType.DMA,
        pltpu.SemaphoreType.DMA,
        pltpu.SemaphoreType.DMA,
    ],
    compiler_params=pltpu.CompilerParams(use_tc_tiling_on_sc=False),
)
def _aggpair_sc(h_hbm, ta_hbm, tb_hbm, row2_hbm, col2_hbm,
                out_hbm, oa_hbm, ob_hbm,
                ridxb, cidxb, rows, pbufa, pbufb, agg_sh, gsem, ssem, wsem):
    cid = lax.axis_index("c")
    sid = lax.axis_index("s")
    w = cid * _NS + sid
    rb = w * _RPE

    _zero_vmem(rows, 128, H)
    _zero_shared_slice(rows, agg_sh, sid)
    plsc.subcore_barrier()

    def fire_g(g, s):
        for k in range(_KA):
            pltpu.async_copy(h_hbm.at[ridxb.at[g * _KA + k]],
                             rows.at[pl.ds((s * _KA + k) * _CH, _CH)], gsem)
            pltpu.async_copy(ta_hbm.at[ridxb.at[g * _KA + k]],
                             pbufa.at[pl.ds((s * _KA + k) * _CH, _CH)], gsem)
            pltpu.async_copy(tb_hbm.at[cidxb.at[g * _KA + k]],
                             pbufb.at[pl.ds((s * _KA + k) * _CH, _CH)], gsem)

    def drain_sw(n):
        _drain(h_hbm.at[pl.ds(0, _CH)], rows.at[pl.ds(0, _CH)], ssem, n)
        _drain(oa_hbm.at[pl.ds(0, _KA * _CH)],
               pbufa.at[pl.ds(0, _KA * _CH)], wsem, 2)

    def superblock(sb, _):
        pltpu.sync_copy(row2_hbm.at[pl.ds(rb + sb * _SB, _SB)], ridxb)
        pltpu.sync_copy(col2_hbm.at[pl.ds(rb + sb * _SB, _SB)], cidxb)
        fire_g(0, 0)

        def group(g, _):
            s = g % 2
            _drain(h_hbm.at[pl.ds(0, _CH)], rows.at[pl.ds(0, _CH)], gsem, _KA)
            _drain(ta_hbm.at[pl.ds(0, _CH)], pbufa.at[pl.ds(0, _CH)],
                   gsem, 2 * _KA)

            @pl.when(g >= 1)
            def _():
                drain_sw(_KA)

            @pl.when(g + 1 < _NGA)
            def _():
                fire_g(g + 1, 1 - s)

            for k in range(_KA):
                pltpu.async_copy(rows.at[pl.ds((s * _KA + k) * _CH, _CH)],
                                 agg_sh.at[cidxb.at[g * _KA + k]], ssem,
                                 add=True)
            e0 = (rb + sb * _SB + g * _KA) * _CH
            pltpu.async_copy(pbufa.at[pl.ds(s * _KA * _CH, _KA * _CH)],
                             oa_hbm.at[pl.ds(e0, _KA * _CH)], wsem)
            pltpu.async_copy(pbufb.at[pl.ds(s * _KA * _CH, _KA * _CH)],
                             ob_hbm.at[pl.ds(e0, _KA * _CH)], wsem)
            return 0

        lax.fori_loop(0, _NGA, group, 0)
        drain_sw(_KA)
        return 0

    lax.fori_loop(0, _NSB, superblock, 0)

    @pl.when(w < _XW)
    def _():
        pltpu.sync_copy(row2_hbm.at[pl.ds(_NW * _RPE + w, 1)],
                        ridxb.at[pl.ds(0, 1)])
        pltpu.sync_copy(col2_hbm.at[pl.ds(_NW * _RPE + w, 1)],
                        cidxb.at[pl.ds(0, 1)])
        gd = [pltpu.async_copy(h_hbm.at[ridxb.at[0]],
                               rows.at[pl.ds(0, _CH)], gsem),
              pltpu.async_copy(ta_hbm.at[ridxb.at[0]],
                               pbufa.at[pl.ds(0, _CH)], gsem),
              pltpu.async_copy(tb_hbm.at[cidxb.at[0]],
                               pbufb.at[pl.ds(0, _CH)], gsem)]
        for d in gd:
            d.wait()
        e0 = (_NW * _RPE + w) * _CH
        sd = [pltpu.async_copy(rows.at[pl.ds(0, _CH)],
                               agg_sh.at[cidxb.at[0]], ssem, add=True),
              pltpu.async_copy(pbufa.at[pl.ds(0, _CH)],
                               oa_hbm.at[pl.ds(e0, _CH)], wsem),
              pltpu.async_copy(pbufb.at[pl.ds(0, _CH)],
                               ob_hbm.at[pl.ds(e0, _CH)], wsem)]
        for d in sd:
            d.wait()

    plsc.subcore_barrier()
    _writeback_slice(agg_sh, out_hbm, cid, sid)


@functools.partial(
    pl.kernel,
    out_type=[jax.ShapeDtypeStruct((E, ED), _f32),
              jax.ShapeDtypeStruct((E, ED), _f32)],
    mesh=_mesh,
    scratch_types=[
        pltpu.VMEM((_RPE + 1, _CH), jnp.int32),
        pltpu.VMEM((_RPE + 1, _CH), jnp.int32),
        pltpu.VMEM((_K * _CH, ED), _f32),
        pltpu.VMEM((_K * _CH, ED), _f32),
        pltpu.SemaphoreType.DMA,
        pltpu.SemaphoreType.DMA,
    ],
    compiler_params=pltpu.CompilerParams(use_tc_tiling_on_sc=False),
)
def _pair_sc(ta_hbm, tb_hbm, ia2_hbm, ib2_hbm, oa_hbm, ob_hbm,
             idxa2, idxb2, bufa, bufb, gsem, wsem):
    cid = lax.axis_index("c")
    sid = lax.axis_index("s")
    w = cid * _NS + sid
    rb = w * _RPE
    base = rb * _CH

    pltpu.sync_copy(ia2_hbm.at[pl.ds(rb, _RPE)], idxa2.at[pl.ds(0, _RPE)])
    pltpu.sync_copy(ib2_hbm.at[pl.ds(rb, _RPE)], idxb2.at[pl.ds(0, _RPE)])

    @pl.when(w < _XW)
    def _():
        pltpu.sync_copy(ia2_hbm.at[pl.ds(_NW * _RPE + w, 1)],
                        idxa2.at[pl.ds(_RPE, 1)])
        pltpu.sync_copy(ib2_hbm.at[pl.ds(_NW * _RPE + w, 1)],
                        idxb2.at[pl.ds(_RPE, 1)])

    def group(g, _):
        gd = [pltpu.async_copy(ta_hbm.at[idxa2.at[g * _K + k]],
                               bufa.at[pl.ds(k * _CH, _CH)], gsem)
              for k in range(_K)]
        gd += [pltpu.async_copy(tb_hbm.at[idxb2.at[g * _K + k]],
                                bufb.at[pl.ds(k * _CH, _CH)], gsem)
               for k in range(_K)]
        for d in gd:
            d.wait()
        e0 = base + g * _K * _CH
        wa = pltpu.async_copy(bufa, oa_hbm.at[pl.ds(e0, _K * _CH)], wsem)
        wb = pltpu.async_copy(bufb, ob_hbm.at[pl.ds(e0, _K * _CH)], wsem)
        wa.wait()
        wb.wait()
        return 0

    lax.fori_loop(0, _NG, group, 0)

    @pl.when(w < _XW)
    def _():
        ga = pltpu.async_copy(ta_hbm.at[idxa2.at[_RPE]],
                              bufa.at[pl.ds(0, _CH)], gsem)
        gb = pltpu.async_copy(tb_hbm.at[idxb2.at[_RPE]],
                              bufb.at[pl.ds(0, _CH)], gsem)
        ga.wait()
        gb.wait()
        e0 = (_NW * _RPE + w) * _CH
        wa = pltpu.async_copy(bufa.at[pl.ds(0, _CH)],
                              oa_hbm.at[pl.ds(e0, _CH)], wsem)
        wb = pltpu.async_copy(bufb.at[pl.ds(0, _CH)],
                              ob_hbm.at[pl.ds(e0, _CH)], wsem)
        wa.wait()
        wb.wait()


_K4 = 4
_NG4 = _RPE // _K4
_X4 = _RPE - _NG4 * _K4


@functools.partial(
    pl.kernel,
    out_type=[jax.ShapeDtypeStruct((E, ED), _f32),
              jax.ShapeDtypeStruct((E, ED), _f32),
              jax.ShapeDtypeStruct((E, ED), _f32),
              jax.ShapeDtypeStruct((E, ED), _f32)],
    mesh=_mesh,
    scratch_types=[
        pltpu.VMEM((_RPE + 1, _CH), jnp.int32),
        pltpu.VMEM((_RPE + 1, _CH), jnp.int32),
        pltpu.VMEM((_K4 * _CH, ED), _f32),
        pltpu.VMEM((_K4 * _CH, ED), _f32),
        pltpu.VMEM((_K4 * _CH, ED), _f32),
        pltpu.VMEM((_K4 * _CH, ED), _f32),
        pltpu.SemaphoreType.DMA,
        pltpu.SemaphoreType.DMA,
    ],
    compiler_params=pltpu.CompilerParams(use_tc_tiling_on_sc=False),
)
def _pair4_sc(ta_hbm, tb_hbm, tp_hbm, ia2_hbm, ib2_hbm,
              oa_hbm, ob_hbm, opr_hbm, opc_hbm,
              idxa2, idxb2, bufa, bufb, bufr, bufc, gsem, wsem):
    cid = lax.axis_index("c")
    sid = lax.axis_index("s")
    w = cid * _NS + sid
    rb = w * _RPE
    base = rb * _CH

    pltpu.sync_copy(ia2_hbm.at[pl.ds(rb, _RPE)], idxa2.at[pl.ds(0, _RPE)])
    pltpu.sync_copy(ib2_hbm.at[pl.ds(rb, _RPE)], idxb2.at[pl.ds(0, _RPE)])

    @pl.when(w < _XW)
    def _():
        pltpu.sync_copy(ia2_hbm.at[pl.ds(_NW * _RPE + w, 1)],
                        idxa2.at[pl.ds(_RPE, 1)])
        pltpu.sync_copy(ib2_hbm.at[pl.ds(_NW * _RPE + w, 1)],
                        idxb2.at[pl.ds(_RPE, 1)])

    def do_chunks(c0, nch, gbase):
        gd = []
        for k in range(nch):
            gd.append(pltpu.async_copy(ta_hbm.at[idxa2.at[gbase + k]],
                                       bufa.at[pl.ds(k * _CH, _CH)], gsem))
            gd.append(pltpu.async_copy(tb_hbm.at[idxb2.at[gbase + k]],
                                       bufb.at[pl.ds(k * _CH, _CH)], gsem))
            gd.append(pltpu.async_copy(tp_hbm.at[idxa2.at[gbase + k]],
                                       bufr.at[pl.ds(k * _CH, _CH)], gsem))
            gd.append(pltpu.async_copy(tp_hbm.at[idxb2.at[gbase + k]],
                                       bufc.at[pl.ds(k * _CH, _CH)], gsem))
        for d in gd:
            d.wait()
        n = nch * _CH
        wd = [pltpu.async_copy(bufa.at[pl.ds(0, n)],
                               oa_hbm.at[pl.ds(c0, n)], wsem),
              pltpu.async_copy(bufb.at[pl.ds(0, n)],
                               ob_hbm.at[pl.ds(c0, n)], wsem),
              pltpu.async_copy(bufr.at[pl.ds(0, n)],
                               opr_hbm.at[pl.ds(c0, n)], wsem),
              pltpu.async_copy(bufc.at[pl.ds(0, n)],
                               opc_hbm.at[pl.ds(c0, n)], wsem)]
        for d in wd:
            d.wait()

    def group(g, _):
        do_chunks(base + g * _K4 * _CH, _K4, g * _K4)
        return 0

    lax.fori_loop(0, _NG4, group, 0)
    do_chunks(base + _NG4 * _K4 * _CH, _X4, _NG4 * _K4)

    @pl.when(w < _XW)
    def _():
        do_chunks((_NW * _RPE + w) * _CH, 1, _RPE)


@functools.partial(
    pl.kernel,
    out_type=jax.ShapeDtypeStruct((_NC, N, ED), _f32),
    mesh=_mesh,
    scratch_types=[
        pltpu.VMEM((_RPE + 1, _CH), jnp.int32),
        pltpu.VMEM((_K * _CH, ED), _f32),
        pltpu.VMEM_SHARED((N, ED), _f32),
        pltpu.SemaphoreType.DMA,
        pltpu.SemaphoreType.DMA,
    ],
    compiler_params=pltpu.CompilerParams(use_tc_tiling_on_sc=False),
)
def _scatter16_sc(vals_hbm, col2_hbm, out_hbm, cidx2, vbuf, acc_sh, lsem, ssem):
    cid = lax.axis_index("c")
    sid = lax.axis_index("s")
    w = cid * _NS + sid
    rb = w * _RPE
    base = rb * _CH

    pltpu.sync_copy(col2_hbm.at[pl.ds(rb, _RPE)], cidx2.at[pl.ds(0, _RPE)])

    @pl.when(w < _XW)
    def _():
        pltpu.sync_copy(col2_hbm.at[pl.ds(_NW * _RPE + w, 1)],
                        cidx2.at[pl.ds(_RPE, 1)])

    _zero_vmem(vbuf, 128, ED)
    _zero_shared_slice(vbuf, acc_sh, sid)
    plsc.subcore_barrier()

    def group(g, _):
        e0 = base + g * _K * _CH
        pltpu.async_copy(vals_hbm.at[pl.ds(e0, _K * _CH)], vbuf, lsem).wait()
        sd = [pltpu.async_copy(vbuf.at[pl.ds(k * _CH, _CH)],
                               acc_sh.at[cidx2.at[g * _K + k]], ssem, add=True)
              for k in range(_K)]
        for d in sd:
            d.wait()
        return 0

    lax.fori_loop(0, _NG, group, 0)

    @pl.when(w < _XW)
    def _():
        e0 = (_NW * _RPE + w) * _CH
        pltpu.async_copy(vals_hbm.at[pl.ds(e0, _CH)],
                         vbuf.at[pl.ds(0, _CH)], lsem).wait()
        pltpu.async_copy(vbuf.at[pl.ds(0, _CH)],
                         acc_sh.at[cidx2.at[_RPE]], ssem, add=True).wait()

    plsc.subcore_barrier()
    _writeback_slice(acc_sh, out_hbm, cid, sid)


_NB = 1000
_GN = N // _NB


def _full(shape):
    return pl.BlockSpec(shape, lambda i: tuple(0 for _ in shape))


def _rows(shape):
    return pl.BlockSpec(shape, lambda i: (i,) + tuple(0 for _ in shape[1:]))


def _embed_body(x_ref, w_ref, b_ref, o_ref):
    o_ref[...] = jnp.dot(x_ref[...], w_ref[...],
                         preferred_element_type=_f32) + b_ref[...]


def _embed(x, w, b):
    return pl.pallas_call(
        _embed_body,
        grid=(_GN,),
        in_specs=[_rows((_NB, H)), _full((H, H)), _full((1, H))],
        out_specs=_rows((_NB, H)),
        out_shape=jax.ShapeDtypeStruct((N, H), _f32),
    )(x, w, b.reshape(1, H))


_P = E // 8
_BP = 4000
_GP = _P // _BP


def _gauss_body(d8_ref, r_ref, off_ref, g_ref, o_ref):
    drep = jnp.dot(d8_ref[...], r_ref[...], preferred_element_type=_f32)
    o_ref[...] = jnp.exp(g_ref[0, 0] * (drep - off_ref[...]) ** 2)


def _gauss(edge_attr, rmat, offs_t, gamma):
    return pl.pallas_call(
        _gauss_body,
        grid=(_GP,),
        in_specs=[_rows((_BP, 8)), _full((8, 128)), _full((1, 128)), _full((1, 1))],
        out_specs=_rows((_BP, 128)),
        out_shape=jax.ShapeDtypeStruct((_P, 128), _f32),
    )(edge_attr.reshape(_P, 8), rmat, offs_t.reshape(1, 128),
      gamma.reshape(1, 1))


def _silu(v):
    return v * jax.nn.sigmoid(v)


def _node_body(h_ref, a0_ref, a1_ref, w1h_ref, w1a_ref, b1_ref, w2_ref,
               b2_ref, g_ref, bb_ref, wr_ref, wc_ref,
               hn_ref, pr_ref, pc_ref):
    h = h_ref[...]
    agg = a0_ref[...] + a1_ref[...]
    z = (jnp.dot(h, w1h_ref[...], preferred_element_type=_f32)
         + jnp.dot(agg, w1a_ref[...], preferred_element_type=_f32)
         + b1_ref[...])
    u = jnp.dot(_silu(z), w2_ref[...], preferred_element_type=_f32) + b2_ref[...]
    hn = h + u
    mean = jnp.mean(hn, axis=-1, keepdims=True)
    d = hn - mean
    var = jnp.mean(d * d, axis=-1, keepdims=True)
    hn = d * lax.rsqrt(var + 1e-5) * g_ref[...] + bb_ref[...]
    hn_ref[...] = hn
    pr_ref[...] = jnp.dot(hn, wr_ref[...], preferred_element_type=_f32)
    pc_ref[...] = jnp.dot(hn, wc_ref[...], preferred_element_type=_f32)


def _node_update(h, a0, a1, w1h, w1a, b1, w2, b2, g, bb, wr, wc):
    return pl.pallas_call(
        _node_body,
        grid=(_GN,),
        in_specs=[_rows((_NB, H)), _rows((_NB, H)), _rows((_NB, H)),
                  _full((H, H)), _full((H, H)), _full((1, H)),
                  _full((H, H)), _full((1, H)), _full((1, H)), _full((1, H)),
                  _full((H, ED)), _full((H, ED))],
        out_specs=[_rows((_NB, H)), _rows((_NB, ED)), _rows((_NB, ED))],
        out_shape=[jax.ShapeDtypeStruct((N, H), _f32),
                   jax.ShapeDtypeStruct((N, ED), _f32),
                   jax.ShapeDtypeStruct((N, ED), _f32)],
    )(h, a0, a1, w1h, w1a, b1.reshape(1, H), w2, b2.reshape(1, H),
      g.reshape(1, H), bb.reshape(1, H), wr, wc)


def _edge_body(ga_ref, gb_ref, ea_ref, we_ref, be1_ref, w2_ref, be2_ref, o_ref):
    ea = ea_ref[...]
    z = (ga_ref[...] + gb_ref[...]
         + jnp.dot(ea, we_ref[...], preferred_element_type=_f32) + be1_ref[...])
    o_ref[...] = ea + jnp.dot(_silu(z), w2_ref[...],
                              preferred_element_type=_f32) + be2_ref[...]


def _edge_mlp(ga, gb, ea, kwe, be1t, kw2, be2t):
    return pl.pallas_call(
        _edge_body,
        grid=(_GP,),
        in_specs=[_rows((_BP, 128)), _rows((_BP, 128)), _rows((_BP, 128)),
                  _full((128, 128)), _full((1, 128)), _full((128, 128)),
                  _full((1, 128))],
        out_specs=_rows((_BP, 128)),
        out_shape=jax.ShapeDtypeStruct((_P, 128), _f32),
    )(ga, gb, ea, kwe, be1t.reshape(1, 128), kw2, be2t.reshape(1, 128))


def _edge0_body(ga_ref, gb_ref, d8_ref, r_ref, off_ref, g_ref,
                we_ref, be1_ref, w2_ref, be2_ref, o_ref):
    drep = jnp.dot(d8_ref[...], r_ref[...], preferred_element_type=_f32)
    ea = jnp.exp(g_ref[0, 0] * (drep - off_ref[...]) ** 2)
    z = (ga_ref[...] + gb_ref[...]
         + jnp.dot(ea, we_ref[...], preferred_element_type=_f32) + be1_ref[...])
    o_ref[...] = ea + jnp.dot(_silu(z), w2_ref[...],
                              preferred_element_type=_f32) + be2_ref[...]


def _edge_mlp0(ga, gb, edge_attr, rmat, offs_t, gamma, kwe, be1t, kw2, be2t):
    return pl.pallas_call(
        _edge0_body,
        grid=(_GP,),
        in_specs=[_rows((_BP, 128)), _rows((_BP, 128)), _rows((_BP, 8)),
                  _full((8, 128)), _full((1, 128)), _full((1, 1)),
                  _full((128, 128)), _full((1, 128)), _full((128, 128)),
                  _full((1, 128))],
        out_specs=_rows((_BP, 128)),
        out_shape=jax.ShapeDtypeStruct((_P, 128), _f32),
    )(ga, gb, edge_attr.reshape(_P, 8), rmat, offs_t.reshape(1, 128),
      gamma.reshape(1, 1), kwe, be1t.reshape(1, 128), kw2,
      be2t.reshape(1, 128))


def _force_body(ea_ref, pr_ref, pc_ref, kw1_ref, b1_ref, kw2_ref, b2_ref,
                s_ref, r_ref, o_ref):
    z = _silu(jnp.dot(ea_ref[...], kw1_ref[...],
                      preferred_element_type=_f32) + b1_ref[...])
    fm8 = jnp.dot(z, kw2_ref[...], preferred_element_type=_f32) + b2_ref[0, 0]
    d = pr_ref[...] - pc_ref[...]
    nrm8 = jnp.sqrt(jnp.dot(d * d, s_ref[...], preferred_element_type=_f32))
    scale = jnp.dot(fm8 / (nrm8 + 1e-8), r_ref[...],
                    preferred_element_type=_f32)
    o_ref[...] = scale * d


def _force(ea, prow, pcol, kw1, b1t, kw2, b2, smat, rmat):
    return pl.pallas_call(
        _force_body,
        grid=(_GP,),
        in_specs=[_rows((_BP, 128)), _rows((_BP, 128)), _rows((_BP, 128)),
                  _full((128, 512)), _full((1, 512)), _full((512, 8)),
                  _full((1, 1)), _full((128, 8)), _full((8, 128))],
        out_specs=_rows((_BP, 128)),
        out_shape=jax.ShapeDtypeStruct((_P, 128), _f32),
    )(ea, prow, pcol, kw1, b1t.reshape(1, 512), kw2, b2.reshape(1, 1),
      smat, rmat)


def _combine_body(p0_ref, p1_ref, o_ref):
    o_ref[...] = (p0_ref[...] + p1_ref[...])[:, :3]


def _combine(p0, p1):
    return pl.pallas_call(
        _combine_body,
        grid=(1,),
        in_specs=[_full((N, ED)), _full((N, ED))],
        out_specs=_full((N, 3)),
        out_shape=jax.ShapeDtypeStruct((N, 3), _f32),
    )(p0, p1)


def kernel(x, pos, edge_index, edge_attr, params):
    row = edge_index[0].astype(jnp.int32)
    col = edge_index[1].astype(jnp.int32)
    row2 = row.reshape(_ECH, _CH)
    col2 = col.reshape(_ECH, _CH)

    h = _embed(x, params['W_ne'], params['b_ne'])

    eye8 = jnp.eye(8, dtype=_f32)
    rmat = jnp.kron(eye8, jnp.ones((1, ED), _f32))
    smat = jnp.kron(eye8, jnp.ones((ED, 1), _f32))
    offs = jnp.linspace(0.0, CUTOFF, ED)
    gamma = -0.5 / (offs[1] - offs[0]) ** 2
    posp = jnp.pad(pos, ((0, 0), (0, ED - 3)))

    layers = params['layers']
    nl = len(layers)
    parts = _agg_sc(h, row2, col2)
    ea = None
    for i, lp in enumerate(layers):
        we1 = lp['We1']
        h, pr, pc = _node_update(
            h, parts[0], parts[1],
            lp['W1'][:H], lp['W1'][H:], lp['b1'], lp['W2'], lp['b2'],
            lp['ln_g'], lp['ln_b'], we1[:H], we1[H:2 * H])
        if i + 1 < nl:
            parts, ga, gb = _aggpair_sc(h, pr, pc, row2, col2)
        else:
            ga, gb, prow, pcol = _pair4_sc(pr, pc, posp, row2, col2)
        kwe = jnp.kron(eye8, we1[2 * H:])
        be1t = jnp.tile(lp['be1'], 8)
        kw2 = jnp.kron(eye8, lp['We2'])
        be2t = jnp.tile(lp['be2'], 8)
        if i == 0:
            ea = _edge_mlp0(ga.reshape(_P, 128), gb.reshape(_P, 128),
                            edge_attr, rmat, jnp.tile(offs, 8), gamma,
                            kwe, be1t, kw2, be2t)
        else:
            ea = _edge_mlp(ga.reshape(_P, 128), gb.reshape(_P, 128), ea,
                           kwe, be1t, kw2, be2t)

    fc = _force(ea, prow.reshape(_P, 128), pcol.reshape(_P, 128),
                jnp.kron(eye8, params['Wr1']), jnp.tile(params['br1'], 8),
                jnp.kron(eye8, params['Wr2']), params['br2'], smat, rmat)
    fparts = _scatter16_sc(fc.reshape(E, ED), col2)
    return _combine(fparts[0], fparts[1])


# speedup vs baseline: 3.0558x; 1.0168x over previous
import functools

import jax
import jax.numpy as jnp
from jax import lax
from jax.experimental import pallas as pl
from jax.experimental.pallas import tpu as pltpu
from jax.experimental.pallas import tpu_sc as plsc

N = 10000
E = 320000
H = 128
ED = 16
CUTOFF = 5.0

_NC = 2
_NS = 16
_NW = _NC * _NS
_CH = 128
_ECH = E // _CH
_RPE = _ECH // _NW
_XW = _ECH - _RPE * _NW
_K = 6
_NG = _RPE // _K
_RW = 624
_RREM = N - _NS * _RW
_ROFF = _NS * _RW

_mesh = plsc.VectorSubcoreMesh(core_axis_name="c", subcore_axis_name="s")
_f32 = jnp.float32


def _zero_vmem(ref, nrows, ncols):
    nv = ncols // 16

    def body(i, _):
        for j in range(nv):
            ref[i, pl.ds(j * 16, 16)] = jnp.zeros((16,), _f32)
        return 0

    lax.fori_loop(0, nrows, body, 0)


def _zero_shared_slice(zbuf, acc_sh, sid):
    r0 = pl.multiple_of(sid * _RW, 8)
    nfull = _RW // 128
    rem = _RW - nfull * 128
    for k in range(nfull):
        pltpu.sync_copy(zbuf.at[pl.ds(0, 128)],
                        acc_sh.at[pl.ds(r0 + k * 128, 128)])
    if rem:
        pltpu.sync_copy(zbuf.at[pl.ds(0, rem)],
                        acc_sh.at[pl.ds(r0 + nfull * 128, rem)])

    @pl.when(sid == _NS - 1)
    def _():
        pltpu.sync_copy(zbuf.at[pl.ds(0, _RREM)], acc_sh.at[pl.ds(_ROFF, _RREM)])


def _writeback_slice(acc_sh, out_hbm, cid, sid):
    r0 = pl.multiple_of(sid * _RW, 8)
    pltpu.sync_copy(acc_sh.at[pl.ds(r0, _RW)], out_hbm.at[cid, pl.ds(r0, _RW)])

    @pl.when(sid == _NS - 1)
    def _():
        pltpu.sync_copy(acc_sh.at[pl.ds(_ROFF, _RREM)],
                        out_hbm.at[cid, pl.ds(_ROFF, _RREM)])


_KA = 1
_SB = 13
_NSB = _RPE // _SB
_NGA = _SB // _KA


def _drain(src_hbm, dst_vmem, sem, n):
    for _ in range(n):
        pltpu.make_async_copy(src_hbm, dst_vmem, sem).wait()


@functools.partial(
    pl.kernel,
    out_type=jax.ShapeDtypeStruct((_NC, N, H), _f32),
    mesh=_mesh,
    scratch_types=[
        pltpu.VMEM((_SB, _CH), jnp.int32),
        pltpu.VMEM((_SB, _CH), jnp.int32),
        pltpu.VMEM((2 * _KA * _CH, H), _f32),
        pltpu.VMEM_SHARED((N, H), _f32),
        pltpu.SemaphoreType.DMA,
        pltpu.SemaphoreType.DMA,
    ],
    compiler_params=pltpu.CompilerParams(use_tc_tiling_on_sc=False),
)
def _agg_sc(h_hbm, row2_hbm, col2_hbm, out_hbm,
            ridxb, cidxb, rows, agg_sh, gsem, ssem):
    cid = lax.axis_index("c")
    sid = lax.axis_index("s")
    w = cid * _NS + sid
    rb = w * _RPE

    _zero_vmem(rows, 128, H)
    _zero_shared_slice(rows, agg_sh, sid)
    plsc.subcore_barrier()

    def fire_g(g, s):
        for k in range(_KA):
            pltpu.async_copy(h_hbm.at[ridxb.at[g * _KA + k]],
                             rows.at[pl.ds((s * _KA + k) * _CH, _CH)], gsem)

    def superblock(sb, _):
        pltpu.sync_copy(row2_hbm.at[pl.ds(rb + sb * _SB, _SB)], ridxb)
        pltpu.sync_copy(col2_hbm.at[pl.ds(rb + sb * _SB, _SB)], cidxb)
        fire_g(0, 0)

        def group(g, _):
            s = g % 2
            _drain(h_hbm.at[pl.ds(0, _CH)], rows.at[pl.ds(0, _CH)], gsem, _KA)

            @pl.when(g >= 1)
            def _():
                _drain(h_hbm.at[pl.ds(0, _CH)], rows.at[pl.ds(0, _CH)],
                       ssem, _KA)

            @pl.when(g + 1 < _NGA)
            def _():
                fire_g(g + 1, 1 - s)

            for k in range(_KA):
                pltpu.async_copy(rows.at[pl.ds((s * _KA + k) * _CH, _CH)],
                                 agg_sh.at[cidxb.at[g * _KA + k]], ssem,
                                 add=True)
            return 0

        lax.fori_loop(0, _NGA, group, 0)
        _drain(h_hbm.at[pl.ds(0, _CH)], rows.at[pl.ds(0, _CH)], ssem, _KA)
        return 0

    lax.fori_loop(0, _NSB, superblock, 0)

    @pl.when(w < _XW)
    def _():
        pltpu.sync_copy(row2_hbm.at[pl.ds(_NW * _RPE + w, 1)],
                        ridxb.at[pl.ds(0, 1)])
        pltpu.sync_copy(col2_hbm.at[pl.ds(_NW * _RPE + w, 1)],
                        cidxb.at[pl.ds(0, 1)])
        pltpu.async_copy(h_hbm.at[ridxb.at[0]],
                         rows.at[pl.ds(0, _CH)], gsem).wait()
        pltpu.async_copy(rows.at[pl.ds(0, _CH)],
                         agg_sh.at[cidxb.at[0]], ssem, add=True).wait()

    plsc.subcore_barrier()
    _writeback_slice(agg_sh, out_hbm, cid, sid)


@functools.partial(
    pl.kernel,
    out_type=[jax.ShapeDtypeStruct((_NC, N, H), _f32),
              jax.ShapeDtypeStruct((E, ED), _f32),
              jax.ShapeDtypeStruct((E, ED), _f32)],
    mesh=_mesh,
    scratch_types=[
        pltpu.VMEM((_SB, _CH), jnp.int32),
        pltpu.VMEM((_SB, _CH), jnp.int32),
        pltpu.VMEM((2 * _KA * _CH, H), _f32),
        pltpu.VMEM((2 * _KA * _CH, ED), _f32),
        pltpu.VMEM((2 * _KA * _CH, ED), _f32),
        pltpu.VMEM_SHARED((N, H), _f32),
        pltpu.SemaphoreType.DMA,
        pltpu.SemaphoreType.DMA,
        pltpu.SemaphoreType.DMA,
    ],
    compiler_params=pltpu.CompilerParams(use_tc_tiling_on_sc=False),
)
def _aggpair_sc(h_hbm, ta_hbm, tb_hbm, row2_hbm, col2_hbm,
                out_hbm, oa_hbm, ob_hbm,
                ridxb, cidxb, rows, pbufa, pbufb, agg_sh, gsem, ssem, wsem):
    cid = lax.axis_index("c")
    sid = lax.axis_index("s")
    w = cid * _NS + sid
    rb = w * _RPE

    _zero_vmem(rows, 128, H)
    _zero_shared_slice(rows, agg_sh, sid)
    plsc.subcore_barrier()

    def fire_g(g, s):
        for k in range(_KA):
            pltpu.async_copy(h_hbm.at[ridxb.at[g * _KA + k]],
                             rows.at[pl.ds((s * _KA + k) * _CH, _CH)], gsem)
            pltpu.async_copy(ta_hbm.at[ridxb.at[g * _KA + k]],
                             pbufa.at[pl.ds((s * _KA + k) * _CH, _CH)], gsem)
            pltpu.async_copy(tb_hbm.at[cidxb.at[g * _KA + k]],
                             pbufb.at[pl.ds((s * _KA + k) * _CH, _CH)], gsem)

    def drain_sw(n):
        _drain(h_hbm.at[pl.ds(0, _CH)], rows.at[pl.ds(0, _CH)], ssem, n)
        _drain(oa_hbm.at[pl.ds(0, _KA * _CH)],
               pbufa.at[pl.ds(0, _KA * _CH)], wsem, 2)

    def superblock(sb, _):
        pltpu.sync_copy(row2_hbm.at[pl.ds(rb + sb * _SB, _SB)], ridxb)
        pltpu.sync_copy(col2_hbm.at[pl.ds(rb + sb * _SB, _SB)], cidxb)
        fire_g(0, 0)

        def group(g, _):
            s = g % 2
            _drain(h_hbm.at[pl.ds(0, _CH)], rows.at[pl.ds(0, _CH)], gsem, _KA)
            _drain(ta_hbm.at[pl.ds(0, _CH)], pbufa.at[pl.ds(0, _CH)],
                   gsem, 2 * _KA)

            @pl.when(g >= 1)
            def _():
                drain_sw(_KA)

            @pl.when(g + 1 < _NGA)
            def _():
                fire_g(g + 1, 1 - s)

            for k in range(_KA):
                pltpu.async_copy(rows.at[pl.ds((s * _KA + k) * _CH, _CH)],
                                 agg_sh.at[cidxb.at[g * _KA + k]], ssem,
                                 add=True)
            e0 = (rb + sb * _SB + g * _KA) * _CH
            pltpu.async_copy(pbufa.at[pl.ds(s * _KA * _CH, _KA * _CH)],
                             oa_hbm.at[pl.ds(e0, _KA * _CH)], wsem)
            pltpu.async_copy(pbufb.at[pl.ds(s * _KA * _CH, _KA * _CH)],
                             ob_hbm.at[pl.ds(e0, _KA * _CH)], wsem)
            return 0

        lax.fori_loop(0, _NGA, group, 0)
        drain_sw(_KA)
        return 0

    lax.fori_loop(0, _NSB, superblock, 0)

    @pl.when(w < _XW)
    def _():
        pltpu.sync_copy(row2_hbm.at[pl.ds(_NW * _RPE + w, 1)],
                        ridxb.at[pl.ds(0, 1)])
        pltpu.sync_copy(col2_hbm.at[pl.ds(_NW * _RPE + w, 1)],
                        cidxb.at[pl.ds(0, 1)])
        gd = [pltpu.async_copy(h_hbm.at[ridxb.at[0]],
                               rows.at[pl.ds(0, _CH)], gsem),
              pltpu.async_copy(ta_hbm.at[ridxb.at[0]],
                               pbufa.at[pl.ds(0, _CH)], gsem),
              pltpu.async_copy(tb_hbm.at[cidxb.at[0]],
                               pbufb.at[pl.ds(0, _CH)], gsem)]
        for d in gd:
            d.wait()
        e0 = (_NW * _RPE + w) * _CH
        sd = [pltpu.async_copy(rows.at[pl.ds(0, _CH)],
                               agg_sh.at[cidxb.at[0]], ssem, add=True),
              pltpu.async_copy(pbufa.at[pl.ds(0, _CH)],
                               oa_hbm.at[pl.ds(e0, _CH)], wsem),
              pltpu.async_copy(pbufb.at[pl.ds(0, _CH)],
                               ob_hbm.at[pl.ds(e0, _CH)], wsem)]
        for d in sd:
            d.wait()

    plsc.subcore_barrier()
    _writeback_slice(agg_sh, out_hbm, cid, sid)


@functools.partial(
    pl.kernel,
    out_type=[jax.ShapeDtypeStruct((E, ED), _f32),
              jax.ShapeDtypeStruct((E, ED), _f32)],
    mesh=_mesh,
    scratch_types=[
        pltpu.VMEM((_RPE + 1, _CH), jnp.int32),
        pltpu.VMEM((_RPE + 1, _CH), jnp.int32),
        pltpu.VMEM((_K * _CH, ED), _f32),
        pltpu.VMEM((_K * _CH, ED), _f32),
        pltpu.SemaphoreType.DMA,
        pltpu.SemaphoreType.DMA,
    ],
    compiler_params=pltpu.CompilerParams(use_tc_tiling_on_sc=False),
)
def _pair_sc(ta_hbm, tb_hbm, ia2_hbm, ib2_hbm, oa_hbm, ob_hbm,
             idxa2, idxb2, bufa, bufb, gsem, wsem):
    cid = lax.axis_index("c")
    sid = lax.axis_index("s")
    w = cid * _NS + sid
    rb = w * _RPE
    base = rb * _CH

    pltpu.sync_copy(ia2_hbm.at[pl.ds(rb, _RPE)], idxa2.at[pl.ds(0, _RPE)])
    pltpu.sync_copy(ib2_hbm.at[pl.ds(rb, _RPE)], idxb2.at[pl.ds(0, _RPE)])

    @pl.when(w < _XW)
    def _():
        pltpu.sync_copy(ia2_hbm.at[pl.ds(_NW * _RPE + w, 1)],
                        idxa2.at[pl.ds(_RPE, 1)])
        pltpu.sync_copy(ib2_hbm.at[pl.ds(_NW * _RPE + w, 1)],
                        idxb2.at[pl.ds(_RPE, 1)])

    def group(g, _):
        gd = [pltpu.async_copy(ta_hbm.at[idxa2.at[g * _K + k]],
                               bufa.at[pl.ds(k * _CH, _CH)], gsem)
              for k in range(_K)]
        gd += [pltpu.async_copy(tb_hbm.at[idxb2.at[g * _K + k]],
                                bufb.at[pl.ds(k * _CH, _CH)], gsem)
               for k in range(_K)]
        for d in gd:
            d.wait()
        e0 = base + g * _K * _CH
        wa = pltpu.async_copy(bufa, oa_hbm.at[pl.ds(e0, _K * _CH)], wsem)
        wb = pltpu.async_copy(bufb, ob_hbm.at[pl.ds(e0, _K * _CH)], wsem)
        wa.wait()
        wb.wait()
        return 0

    lax.fori_loop(0, _NG, group, 0)

    @pl.when(w < _XW)
    def _():
        ga = pltpu.async_copy(ta_hbm.at[idxa2.at[_RPE]],
                              bufa.at[pl.ds(0, _CH)], gsem)
        gb = pltpu.async_copy(tb_hbm.at[idxb2.at[_RPE]],
                              bufb.at[pl.ds(0, _CH)], gsem)
        ga.wait()
        gb.wait()
        e0 = (_NW * _RPE + w) * _CH
        wa = pltpu.async_copy(bufa.at[pl.ds(0, _CH)],
                              oa_hbm.at[pl.ds(e0, _CH)], wsem)
        wb = pltpu.async_copy(bufb.at[pl.ds(0, _CH)],
                              ob_hbm.at[pl.ds(e0, _CH)], wsem)
        wa.wait()
        wb.wait()


_K4 = 4
_NG4 = _RPE // _K4
_X4 = _RPE - _NG4 * _K4


@functools.partial(
    pl.kernel,
    out_type=[jax.ShapeDtypeStruct((E, ED), _f32),
              jax.ShapeDtypeStruct((E, ED), _f32),
              jax.ShapeDtypeStruct((E, ED), _f32),
              jax.ShapeDtypeStruct((E, ED), _f32)],
    mesh=_mesh,
    scratch_types=[
        pltpu.VMEM((_RPE + 1, _CH), jnp.int32),
        pltpu.VMEM((_RPE + 1, _CH), jnp.int32),
        pltpu.VMEM((_K4 * _CH, ED), _f32),
        pltpu.VMEM((_K4 * _CH, ED), _f32),
        pltpu.VMEM((_K4 * _CH, ED), _f32),
        pltpu.VMEM((_K4 * _CH, ED), _f32),
        pltpu.SemaphoreType.DMA,
        pltpu.SemaphoreType.DMA,
    ],
    compiler_params=pltpu.CompilerParams(use_tc_tiling_on_sc=False),
)
def _pair4_sc(ta_hbm, tb_hbm, tp_hbm, ia2_hbm, ib2_hbm,
              oa_hbm, ob_hbm, opr_hbm, opc_hbm,
              idxa2, idxb2, bufa, bufb, bufr, bufc, gsem, wsem):
    cid = lax.axis_index("c")
    sid = lax.axis_index("s")
    w = cid * _NS + sid
    rb = w * _RPE
    base = rb * _CH

    pltpu.sync_copy(ia2_hbm.at[pl.ds(rb, _RPE)], idxa2.at[pl.ds(0, _RPE)])
    pltpu.sync_copy(ib2_hbm.at[pl.ds(rb, _RPE)], idxb2.at[pl.ds(0, _RPE)])

    @pl.when(w < _XW)
    def _():
        pltpu.sync_copy(ia2_hbm.at[pl.ds(_NW * _RPE + w, 1)],
                        idxa2.at[pl.ds(_RPE, 1)])
        pltpu.sync_copy(ib2_hbm.at[pl.ds(_NW * _RPE + w, 1)],
                        idxb2.at[pl.ds(_RPE, 1)])

    def do_chunks(c0, nch, gbase):
        gd = []
        for k in range(nch):
            gd.append(pltpu.async_copy(ta_hbm.at[idxa2.at[gbase + k]],
                                       bufa.at[pl.ds(k * _CH, _CH)], gsem))
            gd.append(pltpu.async_copy(tb_hbm.at[idxb2.at[gbase + k]],
                                       bufb.at[pl.ds(k * _CH, _CH)], gsem))
            gd.append(pltpu.async_copy(tp_hbm.at[idxa2.at[gbase + k]],
                                       bufr.at[pl.ds(k * _CH, _CH)], gsem))
            gd.append(pltpu.async_copy(tp_hbm.at[idxb2.at[gbase + k]],
                                       bufc.at[pl.ds(k * _CH, _CH)], gsem))
        for d in gd:
            d.wait()
        n = nch * _CH
        wd = [pltpu.async_copy(bufa.at[pl.ds(0, n)],
                               oa_hbm.at[pl.ds(c0, n)], wsem),
              pltpu.async_copy(bufb.at[pl.ds(0, n)],
                               ob_hbm.at[pl.ds(c0, n)], wsem),
              pltpu.async_copy(bufr.at[pl.ds(0, n)],
                               opr_hbm.at[pl.ds(c0, n)], wsem),
              pltpu.async_copy(bufc.at[pl.ds(0, n)],
                               opc_hbm.at[pl.ds(c0, n)], wsem)]
        for d in wd:
            d.wait()

    def group(g, _):
        do_chunks(base + g * _K4 * _CH, _K4, g * _K4)
        return 0

    lax.fori_loop(0, _NG4, group, 0)
    do_chunks(base + _NG4 * _K4 * _CH, _X4, _NG4 * _K4)

    @pl.when(w < _XW)
    def _():
        do_chunks((_NW * _RPE + w) * _CH, 1, _RPE)


@functools.partial(
    pl.kernel,
    out_type=jax.ShapeDtypeStruct((_NC, N, ED), _f32),
    mesh=_mesh,
    scratch_types=[
        pltpu.VMEM((_RPE + 1, _CH), jnp.int32),
        pltpu.VMEM((_K * _CH, ED), _f32),
        pltpu.VMEM_SHARED((N, ED), _f32),
        pltpu.SemaphoreType.DMA,
        pltpu.SemaphoreType.DMA,
    ],
    compiler_params=pltpu.CompilerParams(use_tc_tiling_on_sc=False),
)
def _scatter16_sc(vals_hbm, col2_hbm, out_hbm, cidx2, vbuf, acc_sh, lsem, ssem):
    cid = lax.axis_index("c")
    sid = lax.axis_index("s")
    w = cid * _NS + sid
    rb = w * _RPE
    base = rb * _CH

    pltpu.sync_copy(col2_hbm.at[pl.ds(rb, _RPE)], cidx2.at[pl.ds(0, _RPE)])

    @pl.when(w < _XW)
    def _():
        pltpu.sync_copy(col2_hbm.at[pl.ds(_NW * _RPE + w, 1)],
                        cidx2.at[pl.ds(_RPE, 1)])

    _zero_vmem(vbuf, 128, ED)
    _zero_shared_slice(vbuf, acc_sh, sid)
    plsc.subcore_barrier()

    def group(g, _):
        e0 = base + g * _K * _CH
        pltpu.async_copy(vals_hbm.at[pl.ds(e0, _K * _CH)], vbuf, lsem).wait()
        sd = [pltpu.async_copy(vbuf.at[pl.ds(k * _CH, _CH)],
                               acc_sh.at[cidx2.at[g * _K + k]], ssem, add=True)
              for k in range(_K)]
        for d in sd:
            d.wait()
        return 0

    lax.fori_loop(0, _NG, group, 0)

    @pl.when(w < _XW)
    def _():
        e0 = (_NW * _RPE + w) * _CH
        pltpu.async_copy(vals_hbm.at[pl.ds(e0, _CH)],
                         vbuf.at[pl.ds(0, _CH)], lsem).wait()
        pltpu.async_copy(vbuf.at[pl.ds(0, _CH)],
                         acc_sh.at[cidx2.at[_RPE]], ssem, add=True).wait()

    plsc.subcore_barrier()
    _writeback_slice(acc_sh, out_hbm, cid, sid)


_NB = 1000
_GN = N // _NB


def _full(shape):
    return pl.BlockSpec(shape, lambda i: tuple(0 for _ in shape))


def _rows(shape):
    return pl.BlockSpec(shape, lambda i: (i,) + tuple(0 for _ in shape[1:]))


def _embed_body(x_ref, w_ref, b_ref, o_ref):
    o_ref[...] = jnp.dot(x_ref[...], w_ref[...],
                         preferred_element_type=_f32) + b_ref[...]


def _embed(x, w, b):
    return pl.pallas_call(
        _embed_body,
        grid=(_GN,),
        in_specs=[_rows((_NB, H)), _full((H, H)), _full((1, H))],
        out_specs=_rows((_NB, H)),
        out_shape=jax.ShapeDtypeStruct((N, H), _f32),
    )(x, w, b.reshape(1, H))


_P = E // 8
_BP = 4000
_GP = _P // _BP


def _gauss_body(d8_ref, r_ref, off_ref, g_ref, o_ref):
    drep = jnp.dot(d8_ref[...], r_ref[...], preferred_element_type=_f32)
    o_ref[...] = jnp.exp(g_ref[0, 0] * (drep - off_ref[...]) ** 2)


def _gauss(edge_attr, rmat, offs_t, gamma):
    return pl.pallas_call(
        _gauss_body,
        grid=(_GP,),
        in_specs=[_rows((_BP, 8)), _full((8, 128)), _full((1, 128)), _full((1, 1))],
        out_specs=_rows((_BP, 128)),
        out_shape=jax.ShapeDtypeStruct((_P, 128), _f32),
    )(edge_attr.reshape(_P, 8), rmat, offs_t.reshape(1, 128),
      gamma.reshape(1, 1))


def _silu(v):
    return v * jax.nn.sigmoid(v)


def _node_compute(h, agg, w1h_ref, w1a_ref, b1_ref, w2_ref, b2_ref,
                  g_ref, bb_ref):
    z = (jnp.dot(h, w1h_ref[...], preferred_element_type=_f32)
         + jnp.dot(agg, w1a_ref[...], preferred_element_type=_f32)
         + b1_ref[...])
    u = jnp.dot(_silu(z), w2_ref[...], preferred_element_type=_f32) + b2_ref[...]
    hn = h + u
    mean = jnp.mean(hn, axis=-1, keepdims=True)
    d = hn - mean
    var = jnp.mean(d * d, axis=-1, keepdims=True)
    return d * lax.rsqrt(var + 1e-5) * g_ref[...] + bb_ref[...]


def _node_body(h_ref, a0_ref, a1_ref, w1h_ref, w1a_ref, b1_ref, w2_ref,
               b2_ref, g_ref, bb_ref, wr_ref, wc_ref,
               hn_ref, pr_ref, pc_ref):
    hn = _node_compute(h_ref[...], a0_ref[...] + a1_ref[...], w1h_ref,
                       w1a_ref, b1_ref, w2_ref, b2_ref, g_ref, bb_ref)
    hn_ref[...] = hn
    pr_ref[...] = jnp.dot(hn, wr_ref[...], preferred_element_type=_f32)
    pc_ref[...] = jnp.dot(hn, wc_ref[...], preferred_element_type=_f32)


_NODE_SPECS = [_rows((_NB, H)), _rows((_NB, H)), _rows((_NB, H)),
               _full((H, H)), _full((H, H)), _full((1, H)),
               _full((H, H)), _full((1, H)), _full((1, H)), _full((1, H)),
               _full((H, ED)), _full((H, ED))]
_NODE_OUT_SPECS = [_rows((_NB, H)), _rows((_NB, ED)), _rows((_NB, ED))]
_NODE_OUT_SHAPE = [jax.ShapeDtypeStruct((N, H), _f32),
                   jax.ShapeDtypeStruct((N, ED), _f32),
                   jax.ShapeDtypeStruct((N, ED), _f32)]


def _node_update(h, a0, a1, w1h, w1a, b1, w2, b2, g, bb, wr, wc):
    return pl.pallas_call(
        _node_body,
        grid=(_GN,),
        in_specs=_NODE_SPECS,
        out_specs=_NODE_OUT_SPECS,
        out_shape=_NODE_OUT_SHAPE,
    )(h, a0, a1, w1h, w1a, b1.reshape(1, H), w2, b2.reshape(1, H),
      g.reshape(1, H), bb.reshape(1, H), wr, wc)


def _edge_body(ga_ref, gb_ref, ea_ref, we_ref, be1_ref, w2_ref, be2_ref, o_ref):
    ea = ea_ref[...]
    z = (ga_ref[...] + gb_ref[...]
         + jnp.dot(ea, we_ref[...], preferred_element_type=_f32) + be1_ref[...])
    o_ref[...] = ea + jnp.dot(_silu(z), w2_ref[...],
                              preferred_element_type=_f32) + be2_ref[...]


def _edge_mlp(ga, gb, ea, kwe, be1t, kw2, be2t):
    return pl.pallas_call(
        _edge_body,
        grid=(_GP,),
        in_specs=[_rows((_BP, 128)), _rows((_BP, 128)), _rows((_BP, 128)),
                  _full((128, 128)), _full((1, 128)), _full((128, 128)),
                  _full((1, 128))],
        out_specs=_rows((_BP, 128)),
        out_shape=jax.ShapeDtypeStruct((_P, 128), _f32),
    )(ga, gb, ea, kwe, be1t.reshape(1, 128), kw2, be2t.reshape(1, 128))


def _edge0_body(ga_ref, gb_ref, d8_ref, r_ref, off_ref, g_ref,
                we_ref, be1_ref, w2_ref, be2_ref, o_ref):
    drep = jnp.dot(d8_ref[...], r_ref[...], preferred_element_type=_f32)
    ea = jnp.exp(g_ref[0, 0] * (drep - off_ref[...]) ** 2)
    z = (ga_ref[...] + gb_ref[...]
         + jnp.dot(ea, we_ref[...], preferred_element_type=_f32) + be1_ref[...])
    o_ref[...] = ea + jnp.dot(_silu(z), w2_ref[...],
                              preferred_element_type=_f32) + be2_ref[...]


def _edge_mlp0(ga, gb, edge_attr, rmat, offs_t, gamma, kwe, be1t, kw2, be2t):
    return pl.pallas_call(
        _edge0_body,
        grid=(_GP,),
        in_specs=[_rows((_BP, 128)), _rows((_BP, 128)), _rows((_BP, 8)),
                  _full((8, 128)), _full((1, 128)), _full((1, 1)),
                  _full((128, 128)), _full((1, 128)), _full((128, 128)),
                  _full((1, 128))],
        out_specs=_rows((_BP, 128)),
        out_shape=jax.ShapeDtypeStruct((_P, 128), _f32),
    )(ga, gb, edge_attr.reshape(_P, 8), rmat, offs_t.reshape(1, 128),
      gamma.reshape(1, 1), kwe, be1t.reshape(1, 128), kw2,
      be2t.reshape(1, 128))


def _edge_compute(ga_ref, gb_ref, ea, kwe_ref, be1_ref, kw2_ref, be2_ref):
    z = (ga_ref[...] + gb_ref[...]
         + jnp.dot(ea, kwe_ref[...], preferred_element_type=_f32)
         + be1_ref[...])
    return ea + jnp.dot(_silu(z), kw2_ref[...],
                        preferred_element_type=_f32) + be2_ref[...]


_EDGE_SPECS = [_rows((_BP, 128)), _rows((_BP, 128)), _rows((_BP, 128)),
               _full((128, 128)), _full((1, 128)), _full((128, 128)),
               _full((1, 128))]


def _layer_body(ga_ref, gb_ref, ea_ref, kwe_ref, be1_ref, kw2_ref, be2_ref,
                h_ref, a0_ref, a1_ref, w1h_ref, w1a_ref, b1_ref, w2_ref,
                b2_ref, g_ref, bb_ref, wr_ref, wc_ref,
                ean_ref, hn_ref, pr_ref, pc_ref):
    ean_ref[...] = _edge_compute(ga_ref, gb_ref, ea_ref[...], kwe_ref,
                                 be1_ref, kw2_ref, be2_ref)
    hn = _node_compute(h_ref[...], a0_ref[...] + a1_ref[...], w1h_ref,
                       w1a_ref, b1_ref, w2_ref, b2_ref, g_ref, bb_ref)
    hn_ref[...] = hn
    pr_ref[...] = jnp.dot(hn, wr_ref[...], preferred_element_type=_f32)
    pc_ref[...] = jnp.dot(hn, wc_ref[...], preferred_element_type=_f32)


def _layer_tc(ga, gb, ea, kwe, be1t, kw2, be2t,
              h, a0, a1, w1h, w1a, b1, w2, b2, g, bb, wr, wc):
    return pl.pallas_call(
        _layer_body,
        grid=(_GP,),
        in_specs=_EDGE_SPECS + _NODE_SPECS,
        out_specs=[_rows((_BP, 128))] + _NODE_OUT_SPECS,
        out_shape=[jax.ShapeDtypeStruct((_P, 128), _f32)] + _NODE_OUT_SHAPE,
    )(ga, gb, ea, kwe, be1t.reshape(1, 128), kw2, be2t.reshape(1, 128),
      h, a0, a1, w1h, w1a, b1.reshape(1, H), w2, b2.reshape(1, H),
      g.reshape(1, H), bb.reshape(1, H), wr, wc)


def _layer0_body(ga_ref, gb_ref, d8_ref, r_ref, off_ref, gam_ref,
                 kwe_ref, be1_ref, kw2_ref, be2_ref,
                 h_ref, a0_ref, a1_ref, w1h_ref, w1a_ref, b1_ref, w2_ref,
                 b2_ref, g_ref, bb_ref, wr_ref, wc_ref,
                 ean_ref, hn_ref, pr_ref, pc_ref):
    drep = jnp.dot(d8_ref[...], r_ref[...], preferred_element_type=_f32)
    ea = jnp.exp(gam_ref[0, 0] * (drep - off_ref[...]) ** 2)
    ean_ref[...] = _edge_compute(ga_ref, gb_ref, ea, kwe_ref, be1_ref,
                                 kw2_ref, be2_ref)
    hn = _node_compute(h_ref[...], a0_ref[...] + a1_ref[...], w1h_ref,
                       w1a_ref, b1_ref, w2_ref, b2_ref, g_ref, bb_ref)
    hn_ref[...] = hn
    pr_ref[...] = jnp.dot(hn, wr_ref[...], preferred_element_type=_f32)
    pc_ref[...] = jnp.dot(hn, wc_ref[...], preferred_element_type=_f32)


def _layer0_tc(ga, gb, edge_attr, rmat, offs_t, gamma, kwe, be1t, kw2, be2t,
               h, a0, a1, w1h, w1a, b1, w2, b2, g, bb, wr, wc):
    return pl.pallas_call(
        _layer0_body,
        grid=(_GP,),
        in_specs=[_rows((_BP, 128)), _rows((_BP, 128)), _rows((_BP, 8)),
                  _full((8, 128)), _full((1, 128)), _full((1, 1)),
                  _full((128, 128)), _full((1, 128)), _full((128, 128)),
                  _full((1, 128))] + _NODE_SPECS,
        out_specs=[_rows((_BP, 128))] + _NODE_OUT_SPECS,
        out_shape=[jax.ShapeDtypeStruct((_P, 128), _f32)] + _NODE_OUT_SHAPE,
    )(ga, gb, edge_attr.reshape(_P, 8), rmat, offs_t.reshape(1, 128),
      gamma.reshape(1, 1), kwe, be1t.reshape(1, 128), kw2,
      be2t.reshape(1, 128),
      h, a0, a1, w1h, w1a, b1.reshape(1, H), w2, b2.reshape(1, H),
      g.reshape(1, H), bb.reshape(1, H), wr, wc)


def _tail_body(ga_ref, gb_ref, ea_ref, kwe_ref, be1_ref, kw2_ref, be2_ref,
               pr_ref, pc_ref, kw1_ref, b1_ref, kw2r_ref, b2_ref,
               s_ref, r_ref, o_ref):
    ean = _edge_compute(ga_ref, gb_ref, ea_ref[...], kwe_ref, be1_ref,
                        kw2_ref, be2_ref)
    z = _silu(jnp.dot(ean, kw1_ref[...],
                      preferred_element_type=_f32) + b1_ref[...])
    fm8 = jnp.dot(z, kw2r_ref[...], preferred_element_type=_f32) + b2_ref[0, 0]
    d = pr_ref[...] - pc_ref[...]
    nrm8 = jnp.sqrt(jnp.dot(d * d, s_ref[...], preferred_element_type=_f32))
    scale = jnp.dot(fm8 / (nrm8 + 1e-8), r_ref[...],
                    preferred_element_type=_f32)
    o_ref[...] = scale * d


def _tail_tc(ga, gb, ea, kwe, be1t, kw2, be2t, prow, pcol,
             kw1, b1t, kw2r, b2, smat, rmat):
    return pl.pallas_call(
        _tail_body,
        grid=(_GP,),
        in_specs=_EDGE_SPECS + [
            _rows((_BP, 128)), _rows((_BP, 128)),
            _full((128, 512)), _full((1, 512)), _full((512, 8)),
            _full((1, 1)), _full((128, 8)), _full((8, 128))],
        out_specs=_rows((_BP, 128)),
        out_shape=jax.ShapeDtypeStruct((_P, 128), _f32),
    )(ga, gb, ea, kwe, be1t.reshape(1, 128), kw2, be2t.reshape(1, 128),
      prow, pcol, kw1, b1t.reshape(1, 512), kw2r, b2.reshape(1, 1),
      smat, rmat)


def _combine_body(p0_ref, p1_ref, o_ref):
    o_ref[...] = (p0_ref[...] + p1_ref[...])[:, :3]


def _combine(p0, p1):
    return pl.pallas_call(
        _combine_body,
        grid=(1,),
        in_specs=[_full((N, ED)), _full((N, ED))],
        out_specs=_full((N, 3)),
        out_shape=jax.ShapeDtypeStruct((N, 3), _f32),
    )(p0, p1)


def kernel(x, pos, edge_index, edge_attr, params):
    row = edge_index[0].astype(jnp.int32)
    col = edge_index[1].astype(jnp.int32)
    row2 = row.reshape(_ECH, _CH)
    col2 = col.reshape(_ECH, _CH)

    h = _embed(x, params['W_ne'], params['b_ne'])

    eye8 = jnp.eye(8, dtype=_f32)
    rmat = jnp.kron(eye8, jnp.ones((1, ED), _f32))
    smat = jnp.kron(eye8, jnp.ones((ED, 1), _f32))
    offs = jnp.linspace(0.0, CUTOFF, ED)
    gamma = -0.5 / (offs[1] - offs[0]) ** 2
    posp = jnp.pad(pos, ((0, 0), (0, ED - 3)))

    layers = params['layers']

    def nargs(lp):
        we1 = lp['We1']
        return (lp['W1'][:H], lp['W1'][H:], lp['b1'], lp['W2'], lp['b2'],
                lp['ln_g'], lp['ln_b'], we1[:H], we1[H:2 * H])

    def eargs(lp):
        return (jnp.kron(eye8, lp['We1'][2 * H:]), jnp.tile(lp['be1'], 8),
                jnp.kron(eye8, lp['We2']), jnp.tile(lp['be2'], 8))

    parts = _agg_sc(h, row2, col2)
    h, pr, pc = _node_update(h, parts[0], parts[1], *nargs(layers[0]))
    ea = None
    for i in range(3):
        parts, ga, gb = _aggpair_sc(h, pr, pc, row2, col2)
        ga = ga.reshape(_P, 128)
        gb = gb.reshape(_P, 128)
        if i == 0:
            ea, h, pr, pc = _layer0_tc(
                ga, gb, edge_attr, rmat, jnp.tile(offs, 8), gamma,
                *eargs(layers[0]), h, parts[0], parts[1], *nargs(layers[1]))
        else:
            ea, h, pr, pc = _layer_tc(
                ga, gb, ea, *eargs(layers[i]),
                h, parts[0], parts[1], *nargs(layers[i + 1]))

    ga, gb, prow, pcol = _pair4_sc(pr, pc, posp, row2, col2)
    fc = _tail_tc(ga.reshape(_P, 128), gb.reshape(_P, 128), ea,
                  *eargs(layers[3]),
                  prow.reshape(_P, 128), pcol.reshape(_P, 128),
                  jnp.kron(eye8, params['Wr1']), jnp.tile(params['br1'], 8),
                  jnp.kron(eye8, params['Wr2']), params['br2'], smat, rmat)
    fparts = _scatter16_sc(fc.reshape(E, ED), col2)
    return _combine(fparts[0], fparts[1])
